# baseline (speedup 1.0000x reference)
import functools

import jax
import jax.numpy as jnp
from jax import lax
from jax.experimental import pallas as pl
from jax.experimental.pallas import tpu as pltpu
from jax.experimental.pallas import tpu_sc as plsc

N_A, N_P, D, H, OUT, G = 50000, 100000, 64, 64, 2, 128
E_W, E_C, E_S = 1000000, 1000000, 50000

C = 10240
RPT = C // 16
NCH_A, NCH_P = 6, 10
NA_P = NCH_A * C
NP_P = NCH_P * C
EB = 256
EDIV = 16 * EB


def _pad_edges(ei, e_pad):
    src = jnp.pad(ei[0], (0, e_pad - ei.shape[1]))
    dst = jnp.pad(ei[1], (0, e_pad - ei.shape[1]), constant_values=-1)
    return src, dst


def _sc_segsum(y, src, dst, nchunk):
    e_pad = src.shape[0]
    nblk = e_pad // EDIV
    n_pass = nchunk // 2
    mesh = plsc.VectorSubcoreMesh(core_axis_name="c", subcore_axis_name="s")

    @functools.partial(
        pl.kernel, mesh=mesh,
        out_type=jax.ShapeDtypeStruct((nchunk * C, 128), jnp.float32),
        scratch_types=[
            pltpu.VMEM((EB,), jnp.int32),
            pltpu.VMEM((EB,), jnp.int32),
            pltpu.VMEM((2, 128), jnp.int32),
            pltpu.VMEM((EB, 128), jnp.float32),
            pltpu.VMEM((16, 128), jnp.float32),
            pltpu.VMEM_SHARED((C + 16, 128), jnp.float32),
            pltpu.SemaphoreType.DMA,
        ],
    )
    def k(y_hbm, src_hbm, dst_hbm, out_hbm, src_v, dst_v, off_v, rows_v,
          zb_v, acc_sh, sem):
        core = lax.axis_index("c")
        sid = lax.axis_index("s")

        def zinit(i, _):
            for j in range(8):
                zb_v[i, pl.ds(16 * j, 16)] = jnp.zeros((16,), jnp.float32)
            return 0
        lax.fori_loop(0, 16, zinit, 0)

        ebase = sid * (e_pad // 16)

        for p in range(n_pass):
            chunk = 2 * p + core
            lo = chunk * C
            base = sid * RPT

            def zacc(i, _):
                pltpu.sync_copy(zb_v, acc_sh.at[pl.ds(base + i * 16, 16)])
                return 0
            lax.fori_loop(0, 40, zacc, 0)

            @pl.when(sid == 0)
            def _():
                pltpu.sync_copy(zb_v.at[pl.ds(0, 16)], acc_sh.at[pl.ds(C, 16)])

            plsc.subcore_barrier()

            def eblk(b, _):
                off = ebase + b * EB
                pltpu.sync_copy(src_hbm.at[pl.ds(off, EB)], src_v)
                pltpu.sync_copy(dst_hbm.at[pl.ds(off, EB)], dst_v)
                for j in range(EB // 16):
                    d = dst_v[pl.ds(16 * j, 16)]
                    m = (d >= lo) & (d < lo + C)
                    off16 = jnp.where(m, d - lo, C)
                    off_v[j // 8, pl.ds(16 * (j % 8), 16)] = off16
                handles = []
                for q in range(EB // 128):
                    handles.append(pltpu.async_copy(
                        y_hbm.at[src_v.at[pl.ds(q * 128, 128)]],
                        rows_v.at[pl.ds(q * 128, 128)], sem))
                for q in range(EB // 128):
                    handles[q].wait()
                    pltpu.sync_copy(rows_v.at[pl.ds(q * 128, 128)],
                                    acc_sh.at[off_v.at[q]], add=True)
                return 0
            lax.fori_loop(0, nblk, eblk, 0)
            plsc.subcore_barrier()

            def wout(i, _):
                pltpu.sync_copy(acc_sh.at[pl.ds(base + i * 128, 128)],
                                out_hbm.at[pl.ds(lo + base + i * 128, 128)])
                return 0
            lax.fori_loop(0, 5, wout, 0)
            plsc.subcore_barrier()

    return k(y, src, dst)


def _dotT(a, b):
    return lax.dot_general(a, b, (((1,), (1,)), ((), ())),
                           preferred_element_type=jnp.float32)


def _aug(y):
    n = y.shape[0]
    return jnp.concatenate(
        [y, jnp.ones((n, 1), jnp.float32), jnp.zeros((n, 63), jnp.float32)],
        axis=1)


def _lin_body(x_ref, w_ref, o_ref):
    o_ref[...] = _aug(_dotT(x_ref[:, :64], w_ref[...]))


def _tc_linear(x, w):
    n = x.shape[0]
    return pl.pallas_call(
        _lin_body,
        grid=(n // 512,),
        in_specs=[pl.BlockSpec((512, 128), lambda i: (i, 0)),
                  pl.BlockSpec((64, 64), lambda i: (0, 0))],
        out_specs=pl.BlockSpec((512, 128), lambda i: (i, 0)),
        out_shape=jax.ShapeDtypeStruct((n, 128), jnp.float32),
    )(x, w)


def _comb2_body(sw_ref, sc_ref, x_ref, wa_ref, wb_ref, ba_ref, bb_ref,
                o_ref):
    invw = 1.0 / jnp.maximum(sw_ref[:, 64:65], 1.0)
    invc = 1.0 / jnp.maximum(sc_ref[:, 64:65], 1.0)
    w = wa_ref[...] + wb_ref[...]
    b = ba_ref[...] + bb_ref[...]
    o_ref[...] = _aug(jnp.maximum(
        sw_ref[:, :64] * invw + sc_ref[:, :64] * invc
        + _dotT(x_ref[:, :64], w) + b, 0.0))


def _tc_combine2(s_w, s_c, x, wa, wb, ba, bb):
    n = x.shape[0]
    return pl.pallas_call(
        _comb2_body,
        grid=(n // 512,),
        in_specs=[pl.BlockSpec((512, 128), lambda i: (i, 0)),
                  pl.BlockSpec((512, 128), lambda i: (i, 0)),
                  pl.BlockSpec((512, 128), lambda i: (i, 0)),
                  pl.BlockSpec((64, 64), lambda i: (0, 0)),
                  pl.BlockSpec((64, 64), lambda i: (0, 0)),
                  pl.BlockSpec((1, 64), lambda i: (0, 0)),
                  pl.BlockSpec((1, 64), lambda i: (0, 0))],
        out_specs=pl.BlockSpec((512, 128), lambda i: (i, 0)),
        out_shape=jax.ShapeDtypeStruct((n, 128), jnp.float32),
    )(s_w, s_c, x, wa, wb, ba, bb)


def _comb1_body(ss_ref, x_ref, w_ref, b_ref, o_ref):
    inv = 1.0 / jnp.maximum(ss_ref[:, 64:65], 1.0)
    o_ref[...] = _aug(jnp.maximum(
        ss_ref[:, :64] * inv + _dotT(x_ref[:, :64], w_ref[...]) + b_ref[...],
        0.0))


def _tc_combine1(s_s, x, w, b):
    n = x.shape[0]
    return pl.pallas_call(
        _comb1_body,
        grid=(n // 512,),
        in_specs=[pl.BlockSpec((512, 128), lambda i: (i, 0)),
                  pl.BlockSpec((512, 128), lambda i: (i, 0)),
                  pl.BlockSpec((64, 64), lambda i: (0, 0)),
                  pl.BlockSpec((1, 64), lambda i: (0, 0))],
        out_specs=pl.BlockSpec((512, 128), lambda i: (i, 0)),
        out_shape=jax.ShapeDtypeStruct((n, 128), jnp.float32),
    )(s_s, x, w, b)


def _pool_body(b_ref, x_ref, os_ref):
    i = pl.program_id(0)

    @pl.when(i == 0)
    def _():
        os_ref[...] = jnp.zeros_like(os_ref)

    b = b_ref[0, 0, :][None, :]
    oh = (jax.lax.broadcasted_iota(jnp.int32, (G, 1024), 0) == b
          ).astype(jnp.float32)
    os_ref[...] += lax.dot_general(oh, x_ref[...], (((1,), (0,)), ((), ())),
                                   preferred_element_type=jnp.float32)


def _tc_pool(x, b3):
    n = x.shape[0]
    return pl.pallas_call(
        _pool_body,
        grid=(n // 1024,),
        in_specs=[pl.BlockSpec((1, 1, 1024), lambda i: (i, 0, 0)),
                  pl.BlockSpec((1024, 128), lambda i: (i, 0))],
        out_specs=pl.BlockSpec((G, 128), lambda i: (0, 0)),
        out_shape=jax.ShapeDtypeStruct((G, 128), jnp.float32),
    )(b3, x)


def _mlp_body(pa_ref, pp_ref, w1_ref, b1_ref, w2_ref, b2_ref, o_ref):
    xa = pa_ref[:, :64] * (1.0 / jnp.maximum(pa_ref[:, 64:65], 1.0))
    xp = pp_ref[:, :64] * (1.0 / jnp.maximum(pp_ref[:, 64:65], 1.0))
    x = jnp.concatenate([xa, xp], axis=1)
    h = jnp.maximum(_dotT(x, w1_ref[...]) + b1_ref[...], 0.0)
    o_ref[...] = _dotT(h, w2_ref[...]) + b2_ref[...]


def _tc_mlp(pa, pp, w1, b1, w2, b2):
    return pl.pallas_call(
        _mlp_body,
        out_shape=jax.ShapeDtypeStruct((G, OUT), jnp.float32),
    )(pa, pp, w1, b1, w2, b2)


def kernel(x_author, x_paper, edge_index_writes, edge_index_cites,
           edge_index_self, batch_author, batch_paper,
           l1_writes_Wl, l1_writes_bl, l1_writes_Wr,
           l1_cites_Wl, l1_cites_bl, l1_cites_Wr,
           l1_self_Wl, l1_self_bl, l1_self_Wr,
           l2_writes_Wl, l2_writes_bl, l2_writes_Wr,
           l2_cites_Wl, l2_cites_bl, l2_cites_Wr,
           l2_self_Wl, l2_self_bl, l2_self_Wr,
           fc1_W, fc1_b, fc2_W, fc2_b):
    xa = jnp.pad(x_author, ((0, NA_P - N_A), (0, 64)))
    xp = jnp.pad(x_paper, ((0, NP_P - N_P), (0, 64)))

    def epad(e):
        return -(-e // EDIV) * EDIV

    src_w, dst_w = _pad_edges(edge_index_writes, epad(E_W))
    src_c, dst_c = _pad_edges(edge_index_cites, epad(E_C))
    src_s, dst_s = _pad_edges(edge_index_self, epad(E_S))

    ba3 = jnp.pad(batch_author, (0, NA_P - N_A), constant_values=G) \
        .reshape(NA_P // 1024, 1, 1024)
    bp3 = jnp.pad(batch_paper, (0, NP_P - N_P), constant_values=G) \
        .reshape(NP_P // 1024, 1, 1024)

    b2 = lambda v: v.reshape(1, -1)

    a_l, p_l = xa, xp
    for (Wlw, blw, Wrw, Wlc, blc, Wrc, Wls, bls, Wrs) in (
            (l1_writes_Wl, l1_writes_bl, l1_writes_Wr,
             l1_cites_Wl, l1_cites_bl, l1_cites_Wr,
             l1_self_Wl, l1_self_bl, l1_self_Wr),
            (l2_writes_Wl, l2_writes_bl, l2_writes_Wr,
             l2_cites_Wl, l2_cites_bl, l2_cites_Wr,
             l2_self_Wl, l2_self_bl, l2_self_Wr)):
        yw = _tc_linear(a_l, Wlw)
        yc = _tc_linear(p_l, Wlc)
        ys = _tc_linear(a_l, Wls)
        s_w = _sc_segsum(yw, src_w, dst_w, NCH_P)
        s_c = _sc_segsum(yc, src_c, dst_c, NCH_P)
        s_s = _sc_segsum(ys, src_s, dst_s, NCH_A)
        p_new = _tc_combine2(s_w, s_c, p_l, Wrw, Wrc, b2(blw), b2(blc))
        a_new = _tc_combine1(s_s, a_l, Wrs, b2(bls))
        a_l, p_l = a_new, p_new

    pa = _tc_pool(a_l, ba3)
    pp = _tc_pool(p_l, bp3)
    return _tc_mlp(pa, pp, fc1_W, b2(fc1_b), fc2_W, b2(fc2_b))


# speedup vs baseline: 3.4867x; 3.4867x over previous
import functools

import jax
import jax.numpy as jnp
from jax import lax
from jax.experimental import pallas as pl
from jax.experimental.pallas import tpu as pltpu
from jax.experimental.pallas import tpu_sc as plsc

N_A, N_P, D, H, OUT, G = 50000, 100000, 64, 64, 2, 128
E_W, E_C, E_S = 1000000, 1000000, 50000

C = 10240
RPT = C // 16
NCH_A, NCH_P = 6, 10
NA_P = NCH_A * C
NP_P = NCH_P * C
EB = 512
EDIV = 16 * EB
BUF = 4096

_DNUMS = lax.GatherDimensionNumbers(
    offset_dims=(), collapsed_slice_dims=(0,), start_index_map=(0,))


def _pad_edges(ei, e_pad):
    src = jnp.pad(ei[0], (0, e_pad - ei.shape[1]))
    dst = jnp.pad(ei[1], (0, e_pad - ei.shape[1]), constant_values=-1)
    return src, dst


def _sc_segsum(y, src, dst, nchunk):
    e_pad = src.shape[0]
    nblk = e_pad // EDIV
    n_pass = nchunk // 2
    mesh = plsc.VectorSubcoreMesh(core_axis_name="c", subcore_axis_name="s")

    @functools.partial(
        pl.kernel, mesh=mesh,
        out_type=jax.ShapeDtypeStruct((nchunk * C, 128), jnp.float32),
        scratch_types=[
            pltpu.VMEM((EB,), jnp.int32),
            pltpu.VMEM((EB,), jnp.int32),
            pltpu.VMEM((BUF + 16,), jnp.int32),
            pltpu.VMEM((BUF + 16,), jnp.int32),
            pltpu.VMEM((128,), jnp.int32),
            pltpu.VMEM((128,), jnp.int32),
            pltpu.VMEM((256, 128), jnp.float32),
            pltpu.VMEM((16, 128), jnp.float32),
            pltpu.VMEM_SHARED((C + 16, 128), jnp.float32),
            pltpu.SemaphoreType.DMA,
        ],
    )
    def k(y_hbm, src_hbm, dst_hbm, out_hbm, src_v, dst_v, srcr_v, dstr_v,
          ixa_v, ixb_v, rows_v, zb_v, acc_sh, sem):
        core = lax.axis_index("c")
        sid = lax.axis_index("s")

        def zinit(i, _):
            for j in range(8):
                zb_v[i, pl.ds(16 * j, 16)] = jnp.zeros((16,), jnp.float32)
            return 0
        lax.fori_loop(0, 16, zinit, 0)

        ebase = sid * (e_pad // 16)
        ii16 = lax.iota(jnp.int32, 16)

        def _lg(x, idx):
            return lax.gather(
                x, idx[:, None], _DNUMS, (1,),
                mode=lax.GatherScatterMode.PROMISE_IN_BOUNDS)

        def fire(nf, tf):
            @pl.when(tf % 2 == 0)
            def _():
                for j in range(8):
                    ixa_v[pl.ds(16 * j, 16)] = \
                        dstr_v[pl.ds(nf * 128 + 16 * j, 16)]

            @pl.when(tf % 2 == 1)
            def _():
                for j in range(8):
                    ixb_v[pl.ds(16 * j, 16)] = \
                        dstr_v[pl.ds(nf * 128 + 16 * j, 16)]

            pltpu.async_copy(
                y_hbm.at[srcr_v.at[pl.ds(nf * 128, 128)]],
                rows_v.at[pl.ds((tf % 2) * 128, 128)], sem)

        def drain_prev(tf):
            par = (tf - 1) % 2
            pltpu.make_async_copy(
                y_hbm.at[pl.ds(0, 128)],
                rows_v.at[pl.ds(par * 128, 128)], sem).wait()

            @pl.when(par == 0)
            def _():
                pltpu.sync_copy(rows_v.at[pl.ds(0, 128)],
                                acc_sh.at[ixa_v], add=True)

            @pl.when(par == 1)
            def _():
                pltpu.sync_copy(rows_v.at[pl.ds(128, 128)],
                                acc_sh.at[ixb_v], add=True)

        for p in range(n_pass):
            chunk = 2 * p + core
            lo = chunk * C
            base = sid * RPT

            def zacc(i, _):
                pltpu.sync_copy(zb_v, acc_sh.at[pl.ds(base + i * 16, 16)])
                return 0
            lax.fori_loop(0, 40, zacc, 0)

            @pl.when(sid == 0)
            def _():
                pltpu.sync_copy(zb_v.at[pl.ds(0, 16)], acc_sh.at[pl.ds(C, 16)])

            plsc.subcore_barrier()

            def eblk(b, carry):
                n, nf, tf = carry
                off = ebase + b * EB
                pltpu.sync_copy(src_hbm.at[pl.ds(off, EB)], src_v)
                pltpu.sync_copy(dst_hbm.at[pl.ds(off, EB)], dst_v)
                for j in range(EB // 16):
                    u = dst_v[pl.ds(16 * j, 16)] - lo
                    m = (u >= 0) & (u < C)
                    mi = jnp.where(m, 1, 0)
                    s = mi
                    for k2 in (1, 2, 4, 8):
                        g = _lg(s, jnp.maximum(ii16 - k2, 0))
                        s = s + jnp.where(ii16 >= k2, g, 0)
                    lo2 = jnp.zeros((16,), jnp.int32)
                    for st in (8, 4, 2, 1):
                        cand = lo2 + st
                        sv = _lg(s, cand - 1)
                        lo2 = jnp.where(sv < ii16 + 1, cand, lo2)
                    lo2 = jnp.minimum(lo2, 15)
                    srcr_v[pl.ds(n, 16)] = _lg(src_v[pl.ds(16 * j, 16)], lo2)
                    dstr_v[pl.ds(n, 16)] = _lg(u, lo2)
                    n = n + s[15]
                for _f in range(EB // 128):
                    @pl.when(nf + _f < n // 128)
                    def _():
                        @pl.when(tf + _f > 0)
                        def _():
                            drain_prev(tf + _f)
                        fire(nf + _f, tf + _f)
                tf = tf + (n // 128 - nf)
                nf = n // 128
                rb = n >= BUF - 768

                @pl.when(rb)
                def _():
                    for j in range(8):
                        sv = srcr_v[pl.ds(nf * 128 + 16 * j, 16)]
                        srcr_v[pl.ds(16 * j, 16)] = sv
                        dv = dstr_v[pl.ds(nf * 128 + 16 * j, 16)]
                        dstr_v[pl.ds(16 * j, 16)] = dv
                n = jnp.where(rb, n - nf * 128, n)
                nf = jnp.where(rb, 0, nf)
                return n, nf, tf
            n, nf, tf = lax.fori_loop(
                0, nblk, eblk,
                (jnp.int32(0), jnp.int32(0), jnp.int32(0)))

            tail = n % 128

            @pl.when(tail > 0)
            def _():
                for j in range(8):
                    srcr_v[pl.ds(n + 16 * j, 16)] = \
                        jnp.zeros((16,), jnp.int32)
                    dstr_v[pl.ds(n + 16 * j, 16)] = \
                        jnp.full((16,), C, jnp.int32)

                @pl.when(tf > 0)
                def _():
                    drain_prev(tf)
                fire(nf, tf)

            @pl.when(tail > 0)
            def _():
                drain_prev(tf + 1)

            @pl.when((tail == 0) & (tf > 0))
            def _():
                drain_prev(tf)

            plsc.subcore_barrier()

            def wout(i, _):
                pltpu.sync_copy(acc_sh.at[pl.ds(base + i * 128, 128)],
                                out_hbm.at[pl.ds(lo + base + i * 128, 128)])
                return 0
            lax.fori_loop(0, 5, wout, 0)
            plsc.subcore_barrier()

    return k(y, src, dst)


def _dotT(a, b):
    return lax.dot_general(a, b, (((1,), (1,)), ((), ())),
                           preferred_element_type=jnp.float32)


def _aug(y):
    n = y.shape[0]
    return jnp.concatenate(
        [y, jnp.ones((n, 1), jnp.float32), jnp.zeros((n, 63), jnp.float32)],
        axis=1)


def _lin_body(x_ref, w_ref, o_ref):
    o_ref[...] = _aug(_dotT(x_ref[:, :64], w_ref[...]))


def _tc_linear(x, w):
    n = x.shape[0]
    return pl.pallas_call(
        _lin_body,
        grid=(n // 512,),
        in_specs=[pl.BlockSpec((512, 128), lambda i: (i, 0)),
                  pl.BlockSpec((64, 64), lambda i: (0, 0))],
        out_specs=pl.BlockSpec((512, 128), lambda i: (i, 0)),
        out_shape=jax.ShapeDtypeStruct((n, 128), jnp.float32),
    )(x, w)


def _comb2_body(sw_ref, sc_ref, x_ref, wa_ref, wb_ref, ba_ref, bb_ref,
                o_ref):
    invw = 1.0 / jnp.maximum(sw_ref[:, 64:65], 1.0)
    invc = 1.0 / jnp.maximum(sc_ref[:, 64:65], 1.0)
    w = wa_ref[...] + wb_ref[...]
    b = ba_ref[...] + bb_ref[...]
    o_ref[...] = _aug(jnp.maximum(
        sw_ref[:, :64] * invw + sc_ref[:, :64] * invc
        + _dotT(x_ref[:, :64], w) + b, 0.0))


def _tc_combine2(s_w, s_c, x, wa, wb, ba, bb):
    n = x.shape[0]
    return pl.pallas_call(
        _comb2_body,
        grid=(n // 512,),
        in_specs=[pl.BlockSpec((512, 128), lambda i: (i, 0)),
                  pl.BlockSpec((512, 128), lambda i: (i, 0)),
                  pl.BlockSpec((512, 128), lambda i: (i, 0)),
                  pl.BlockSpec((64, 64), lambda i: (0, 0)),
                  pl.BlockSpec((64, 64), lambda i: (0, 0)),
                  pl.BlockSpec((1, 64), lambda i: (0, 0)),
                  pl.BlockSpec((1, 64), lambda i: (0, 0))],
        out_specs=pl.BlockSpec((512, 128), lambda i: (i, 0)),
        out_shape=jax.ShapeDtypeStruct((n, 128), jnp.float32),
    )(s_w, s_c, x, wa, wb, ba, bb)


def _comb1_body(ss_ref, x_ref, w_ref, b_ref, o_ref):
    inv = 1.0 / jnp.maximum(ss_ref[:, 64:65], 1.0)
    o_ref[...] = _aug(jnp.maximum(
        ss_ref[:, :64] * inv + _dotT(x_ref[:, :64], w_ref[...]) + b_ref[...],
        0.0))


def _tc_combine1(s_s, x, w, b):
    n = x.shape[0]
    return pl.pallas_call(
        _comb1_body,
        grid=(n // 512,),
        in_specs=[pl.BlockSpec((512, 128), lambda i: (i, 0)),
                  pl.BlockSpec((512, 128), lambda i: (i, 0)),
                  pl.BlockSpec((64, 64), lambda i: (0, 0)),
                  pl.BlockSpec((1, 64), lambda i: (0, 0))],
        out_specs=pl.BlockSpec((512, 128), lambda i: (i, 0)),
        out_shape=jax.ShapeDtypeStruct((n, 128), jnp.float32),
    )(s_s, x, w, b)


def _pool_body(b_ref, x_ref, os_ref):
    i = pl.program_id(0)

    @pl.when(i == 0)
    def _():
        os_ref[...] = jnp.zeros_like(os_ref)

    b = b_ref[0, 0, :][None, :]
    oh = (jax.lax.broadcasted_iota(jnp.int32, (G, 1024), 0) == b
          ).astype(jnp.float32)
    os_ref[...] += lax.dot_general(oh, x_ref[...], (((1,), (0,)), ((), ())),
                                   preferred_element_type=jnp.float32)


def _tc_pool(x, b3):
    n = x.shape[0]
    return pl.pallas_call(
        _pool_body,
        grid=(n // 1024,),
        in_specs=[pl.BlockSpec((1, 1, 1024), lambda i: (i, 0, 0)),
                  pl.BlockSpec((1024, 128), lambda i: (i, 0))],
        out_specs=pl.BlockSpec((G, 128), lambda i: (0, 0)),
        out_shape=jax.ShapeDtypeStruct((G, 128), jnp.float32),
    )(b3, x)


def _mlp_body(pa_ref, pp_ref, w1_ref, b1_ref, w2_ref, b2_ref, o_ref):
    xa = pa_ref[:, :64] * (1.0 / jnp.maximum(pa_ref[:, 64:65], 1.0))
    xp = pp_ref[:, :64] * (1.0 / jnp.maximum(pp_ref[:, 64:65], 1.0))
    x = jnp.concatenate([xa, xp], axis=1)
    h = jnp.maximum(_dotT(x, w1_ref[...]) + b1_ref[...], 0.0)
    o_ref[...] = _dotT(h, w2_ref[...]) + b2_ref[...]


def _tc_mlp(pa, pp, w1, b1, w2, b2):
    return pl.pallas_call(
        _mlp_body,
        out_shape=jax.ShapeDtypeStruct((G, OUT), jnp.float32),
    )(pa, pp, w1, b1, w2, b2)


def kernel(x_author, x_paper, edge_index_writes, edge_index_cites,
           edge_index_self, batch_author, batch_paper,
           l1_writes_Wl, l1_writes_bl, l1_writes_Wr,
           l1_cites_Wl, l1_cites_bl, l1_cites_Wr,
           l1_self_Wl, l1_self_bl, l1_self_Wr,
           l2_writes_Wl, l2_writes_bl, l2_writes_Wr,
           l2_cites_Wl, l2_cites_bl, l2_cites_Wr,
           l2_self_Wl, l2_self_bl, l2_self_Wr,
           fc1_W, fc1_b, fc2_W, fc2_b):
    xa = jnp.pad(x_author, ((0, NA_P - N_A), (0, 64)))
    xp = jnp.pad(x_paper, ((0, NP_P - N_P), (0, 64)))

    def epad(e):
        return -(-e // EDIV) * EDIV

    src_w, dst_w = _pad_edges(edge_index_writes, epad(E_W))
    src_c, dst_c = _pad_edges(edge_index_cites, epad(E_C))
    src_s, dst_s = _pad_edges(edge_index_self, epad(E_S))

    ba3 = jnp.pad(batch_author, (0, NA_P - N_A), constant_values=G) \
        .reshape(NA_P // 1024, 1, 1024)
    bp3 = jnp.pad(batch_paper, (0, NP_P - N_P), constant_values=G) \
        .reshape(NP_P // 1024, 1, 1024)

    b2 = lambda v: v.reshape(1, -1)

    a_l, p_l = xa, xp
    for (Wlw, blw, Wrw, Wlc, blc, Wrc, Wls, bls, Wrs) in (
            (l1_writes_Wl, l1_writes_bl, l1_writes_Wr,
             l1_cites_Wl, l1_cites_bl, l1_cites_Wr,
             l1_self_Wl, l1_self_bl, l1_self_Wr),
            (l2_writes_Wl, l2_writes_bl, l2_writes_Wr,
             l2_cites_Wl, l2_cites_bl, l2_cites_Wr,
             l2_self_Wl, l2_self_bl, l2_self_Wr)):
        yw = _tc_linear(a_l, Wlw)
        yc = _tc_linear(p_l, Wlc)
        ys = _tc_linear(a_l, Wls)
        s_w = _sc_segsum(yw, src_w, dst_w, NCH_P)
        s_c = _sc_segsum(yc, src_c, dst_c, NCH_P)
        s_s = _sc_segsum(ys, src_s, dst_s, NCH_A)
        p_new = _tc_combine2(s_w, s_c, p_l, Wrw, Wrc, b2(blw), b2(blc))
        a_new = _tc_combine1(s_s, a_l, Wrs, b2(bls))
        a_l, p_l = a_new, p_new

    pa = _tc_pool(a_l, ba3)
    pp = _tc_pool(p_l, bp3)
    return _tc_mlp(pa, pp, fc1_W, b2(fc1_b), fc2_W, b2(fc2_b))


# speedup vs baseline: 5.1112x; 1.4659x over previous
import functools

import jax
import jax.numpy as jnp
from jax import lax
from jax.experimental import pallas as pl
from jax.experimental.pallas import tpu as pltpu
from jax.experimental.pallas import tpu_sc as plsc

N_A, N_P, D, H, OUT, G = 50000, 100000, 64, 64, 2, 128
E_W, E_C, E_S = 1000000, 1000000, 50000

C = 10240
RPT = C // 16
NCH_A, NCH_P = 6, 10
NA_P = NCH_A * C
NP_P = NCH_P * C
EB = 1024
EDIV = 16 * EB
BUF = 4096
RBT = BUF - EB - 128

_DNUMS = lax.GatherDimensionNumbers(
    offset_dims=(), collapsed_slice_dims=(0,), start_index_map=(0,))


def _pad_edges(ei, e_pad):
    src = jnp.pad(ei[0], (0, e_pad - ei.shape[1]))
    dst = jnp.pad(ei[1], (0, e_pad - ei.shape[1]), constant_values=-1)
    return src, dst


def _sc_segsum(y, src, dst, nchunk):
    e_pad = src.shape[0]
    nblk = e_pad // EDIV
    n_pass = nchunk // 2
    mesh = plsc.VectorSubcoreMesh(core_axis_name="c", subcore_axis_name="s")

    @functools.partial(
        pl.kernel, mesh=mesh,
        out_type=jax.ShapeDtypeStruct((nchunk * C, 128), jnp.float32),
        scratch_types=[
            pltpu.VMEM((2 * EB,), jnp.int32),
            pltpu.VMEM((2 * EB,), jnp.int32),
            pltpu.VMEM((BUF + 16,), jnp.int32),
            pltpu.VMEM((BUF + 16,), jnp.int32),
            pltpu.VMEM((128,), jnp.int32),
            pltpu.VMEM((128,), jnp.int32),
            pltpu.VMEM((256, 128), jnp.float32),
            pltpu.VMEM_SHARED((C + 16, 128), jnp.float32),
            pltpu.SemaphoreType.DMA,
            pltpu.SemaphoreType.DMA,
            pltpu.SemaphoreType.DMA,
        ],
    )
    def k(y_hbm, src_hbm, dst_hbm, out_hbm, src_v, dst_v, srcr_v, dstr_v,
          ixa_v, ixb_v, rows_v, acc_sh, sem, sem_e, sem_o):
        core = lax.axis_index("c")
        sid = lax.axis_index("s")

        ebase = sid * (e_pad // 16)
        ii16 = lax.iota(jnp.int32, 16)

        def _lg(x, idx):
            return lax.gather(
                x, idx[:, None], _DNUMS, (1,),
                mode=lax.GatherScatterMode.PROMISE_IN_BOUNDS)

        def fire(nf, tf):
            @pl.when(tf % 2 == 0)
            def _():
                for j in range(8):
                    ixa_v[pl.ds(16 * j, 16)] = \
                        dstr_v[pl.ds(nf * 128 + 16 * j, 16)]

            @pl.when(tf % 2 == 1)
            def _():
                for j in range(8):
                    ixb_v[pl.ds(16 * j, 16)] = \
                        dstr_v[pl.ds(nf * 128 + 16 * j, 16)]

            pltpu.async_copy(
                y_hbm.at[srcr_v.at[pl.ds(nf * 128, 128)]],
                rows_v.at[pl.ds((tf % 2) * 128, 128)], sem)

        def drain_prev(tf):
            par = (tf - 1) % 2
            pltpu.make_async_copy(
                y_hbm.at[pl.ds(0, 128)],
                rows_v.at[pl.ds(par * 128, 128)], sem).wait()

            @pl.when(par == 0)
            def _():
                pltpu.sync_copy(rows_v.at[pl.ds(0, 128)],
                                acc_sh.at[ixa_v], add=True)

            @pl.when(par == 1)
            def _():
                pltpu.sync_copy(rows_v.at[pl.ds(128, 128)],
                                acc_sh.at[ixb_v], add=True)

        for p in range(n_pass):
            chunk = 2 * p + core
            lo = chunk * C
            base = sid * RPT

            def zrows(i, _):
                for j in range(8):
                    rows_v[i, pl.ds(16 * j, 16)] = jnp.zeros(
                        (16,), jnp.float32)
                return 0
            lax.fori_loop(0, 256, zrows, 0)
            pltpu.sync_copy(rows_v, acc_sh.at[pl.ds(base, 256)])
            pltpu.sync_copy(rows_v, acc_sh.at[pl.ds(base + 256, 256)])
            pltpu.sync_copy(rows_v.at[pl.ds(0, 128)],
                            acc_sh.at[pl.ds(base + 512, 128)])

            @pl.when(sid == 0)
            def _():
                pltpu.sync_copy(rows_v.at[pl.ds(0, 16)],
                                acc_sh.at[pl.ds(C, 16)])

            plsc.subcore_barrier()

            pltpu.async_copy(src_hbm.at[pl.ds(ebase, EB)],
                             src_v.at[pl.ds(0, EB)], sem_e)
            pltpu.async_copy(dst_hbm.at[pl.ds(ebase, EB)],
                             dst_v.at[pl.ds(0, EB)], sem_e)

            def eblk(b, carry):
                n, nf, tf = carry
                nxt = b + 1

                @pl.when((nxt < nblk) & (nxt % 2 == 0))
                def _():
                    pltpu.async_copy(src_hbm.at[pl.ds(ebase + nxt * EB, EB)],
                                     src_v.at[pl.ds(0, EB)], sem_e)
                    pltpu.async_copy(dst_hbm.at[pl.ds(ebase + nxt * EB, EB)],
                                     dst_v.at[pl.ds(0, EB)], sem_e)

                @pl.when((nxt < nblk) & (nxt % 2 == 1))
                def _():
                    pltpu.async_copy(src_hbm.at[pl.ds(ebase + nxt * EB, EB)],
                                     src_v.at[pl.ds(EB, EB)], sem_o)
                    pltpu.async_copy(dst_hbm.at[pl.ds(ebase + nxt * EB, EB)],
                                     dst_v.at[pl.ds(EB, EB)], sem_o)

                @pl.when(b % 2 == 0)
                def _():
                    pltpu.make_async_copy(src_hbm.at[pl.ds(0, EB)],
                                          src_v.at[pl.ds(0, EB)],
                                          sem_e).wait()
                    pltpu.make_async_copy(src_hbm.at[pl.ds(0, EB)],
                                          dst_v.at[pl.ds(0, EB)],
                                          sem_e).wait()

                @pl.when(b % 2 == 1)
                def _():
                    pltpu.make_async_copy(src_hbm.at[pl.ds(0, EB)],
                                          src_v.at[pl.ds(EB, EB)],
                                          sem_o).wait()
                    pltpu.make_async_copy(src_hbm.at[pl.ds(0, EB)],
                                          dst_v.at[pl.ds(EB, EB)],
                                          sem_o).wait()

                bb = (b % 2) * EB
                def vbody(j, n):
                    u = dst_v[pl.ds(bb + 16 * j, 16)] - lo
                    m = (u >= 0) & (u < C)
                    mi = jnp.where(m, 1, 0)
                    s = mi
                    for k2 in (1, 2, 4, 8):
                        g = _lg(s, jnp.maximum(ii16 - k2, 0))
                        s = s + jnp.where(ii16 >= k2, g, 0)
                    lo2 = jnp.zeros((16,), jnp.int32)
                    for st in (8, 4, 2, 1):
                        cand = lo2 + st
                        sv = _lg(s, cand - 1)
                        lo2 = jnp.where(sv < ii16 + 1, cand, lo2)
                    lo2 = jnp.minimum(lo2, 15)
                    srcr_v[pl.ds(n, 16)] = _lg(
                        src_v[pl.ds(bb + 16 * j, 16)], lo2)
                    dstr_v[pl.ds(n, 16)] = _lg(u, lo2)
                    return n + s[15]
                n = lax.fori_loop(0, EB // 16, vbody, n)
                for _f in range(EB // 128):
                    @pl.when(nf + _f < n // 128)
                    def _():
                        @pl.when(tf + _f > 0)
                        def _():
                            drain_prev(tf + _f)
                        fire(nf + _f, tf + _f)
                tf = tf + (n // 128 - nf)
                nf = n // 128
                rb = n >= RBT

                @pl.when(rb)
                def _():
                    for j in range(8):
                        sv = srcr_v[pl.ds(nf * 128 + 16 * j, 16)]
                        srcr_v[pl.ds(16 * j, 16)] = sv
                        dv = dstr_v[pl.ds(nf * 128 + 16 * j, 16)]
                        dstr_v[pl.ds(16 * j, 16)] = dv
                n = jnp.where(rb, n - nf * 128, n)
                nf = jnp.where(rb, 0, nf)
                return n, nf, tf
            n, nf, tf = lax.fori_loop(
                0, nblk, eblk,
                (jnp.int32(0), jnp.int32(0), jnp.int32(0)))

            tail = n % 128

            @pl.when(tail > 0)
            def _():
                for j in range(8):
                    srcr_v[pl.ds(n + 16 * j, 16)] = \
                        jnp.zeros((16,), jnp.int32)
                    dstr_v[pl.ds(n + 16 * j, 16)] = \
                        jnp.full((16,), C, jnp.int32)

                @pl.when(tf > 0)
                def _():
                    drain_prev(tf)
                fire(nf, tf)

            @pl.when(tail > 0)
            def _():
                drain_prev(tf + 1)

            @pl.when((tail == 0) & (tf > 0))
            def _():
                drain_prev(tf)

            plsc.subcore_barrier()

            def wout(i, _):
                pltpu.sync_copy(acc_sh.at[pl.ds(base + i * 128, 128)],
                                out_hbm.at[pl.ds(lo + base + i * 128, 128)])
                return 0
            lax.fori_loop(0, 5, wout, 0)
            plsc.subcore_barrier()

    return k(y, src, dst)


def _dotT(a, b):
    return lax.dot_general(a, b, (((1,), (1,)), ((), ())),
                           preferred_element_type=jnp.float32)


def _aug(y):
    n = y.shape[0]
    return jnp.concatenate(
        [y, jnp.ones((n, 1), jnp.float32), jnp.zeros((n, 63), jnp.float32)],
        axis=1)


def _lin_body(x_ref, w_ref, o_ref):
    o_ref[...] = _aug(_dotT(x_ref[:, :64], w_ref[...]))


def _tc_linear(x, w):
    n = x.shape[0]
    return pl.pallas_call(
        _lin_body,
        grid=(n // 512,),
        in_specs=[pl.BlockSpec((512, 128), lambda i: (i, 0)),
                  pl.BlockSpec((64, 64), lambda i: (0, 0))],
        out_specs=pl.BlockSpec((512, 128), lambda i: (i, 0)),
        out_shape=jax.ShapeDtypeStruct((n, 128), jnp.float32),
    )(x, w)


def _comb2_body(sw_ref, sc_ref, x_ref, wa_ref, wb_ref, ba_ref, bb_ref,
                o_ref):
    invw = 1.0 / jnp.maximum(sw_ref[:, 64:65], 1.0)
    invc = 1.0 / jnp.maximum(sc_ref[:, 64:65], 1.0)
    w = wa_ref[...] + wb_ref[...]
    b = ba_ref[...] + bb_ref[...]
    o_ref[...] = _aug(jnp.maximum(
        sw_ref[:, :64] * invw + sc_ref[:, :64] * invc
        + _dotT(x_ref[:, :64], w) + b, 0.0))


def _tc_combine2(s_w, s_c, x, wa, wb, ba, bb):
    n = x.shape[0]
    return pl.pallas_call(
        _comb2_body,
        grid=(n // 512,),
        in_specs=[pl.BlockSpec((512, 128), lambda i: (i, 0)),
                  pl.BlockSpec((512, 128), lambda i: (i, 0)),
                  pl.BlockSpec((512, 128), lambda i: (i, 0)),
                  pl.BlockSpec((64, 64), lambda i: (0, 0)),
                  pl.BlockSpec((64, 64), lambda i: (0, 0)),
                  pl.BlockSpec((1, 64), lambda i: (0, 0)),
                  pl.BlockSpec((1, 64), lambda i: (0, 0))],
        out_specs=pl.BlockSpec((512, 128), lambda i: (i, 0)),
        out_shape=jax.ShapeDtypeStruct((n, 128), jnp.float32),
    )(s_w, s_c, x, wa, wb, ba, bb)


def _comb1_body(ss_ref, x_ref, w_ref, b_ref, o_ref):
    inv = 1.0 / jnp.maximum(ss_ref[:, 64:65], 1.0)
    o_ref[...] = _aug(jnp.maximum(
        ss_ref[:, :64] * inv + _dotT(x_ref[:, :64], w_ref[...]) + b_ref[...],
        0.0))


def _tc_combine1(s_s, x, w, b):
    n = x.shape[0]
    return pl.pallas_call(
        _comb1_body,
        grid=(n // 512,),
        in_specs=[pl.BlockSpec((512, 128), lambda i: (i, 0)),
                  pl.BlockSpec((512, 128), lambda i: (i, 0)),
                  pl.BlockSpec((64, 64), lambda i: (0, 0)),
                  pl.BlockSpec((1, 64), lambda i: (0, 0))],
        out_specs=pl.BlockSpec((512, 128), lambda i: (i, 0)),
        out_shape=jax.ShapeDtypeStruct((n, 128), jnp.float32),
    )(s_s, x, w, b)


def _pool_body(b_ref, x_ref, os_ref):
    i = pl.program_id(0)

    @pl.when(i == 0)
    def _():
        os_ref[...] = jnp.zeros_like(os_ref)

    b = b_ref[0, 0, :][None, :]
    oh = (jax.lax.broadcasted_iota(jnp.int32, (G, 1024), 0) == b
          ).astype(jnp.float32)
    os_ref[...] += lax.dot_general(oh, x_ref[...], (((1,), (0,)), ((), ())),
                                   preferred_element_type=jnp.float32)


def _tc_pool(x, b3):
    n = x.shape[0]
    return pl.pallas_call(
        _pool_body,
        grid=(n // 1024,),
        in_specs=[pl.BlockSpec((1, 1, 1024), lambda i: (i, 0, 0)),
                  pl.BlockSpec((1024, 128), lambda i: (i, 0))],
        out_specs=pl.BlockSpec((G, 128), lambda i: (0, 0)),
        out_shape=jax.ShapeDtypeStruct((G, 128), jnp.float32),
    )(b3, x)


def _mlp_body(pa_ref, pp_ref, w1_ref, b1_ref, w2_ref, b2_ref, o_ref):
    xa = pa_ref[:, :64] * (1.0 / jnp.maximum(pa_ref[:, 64:65], 1.0))
    xp = pp_ref[:, :64] * (1.0 / jnp.maximum(pp_ref[:, 64:65], 1.0))
    x = jnp.concatenate([xa, xp], axis=1)
    h = jnp.maximum(_dotT(x, w1_ref[...]) + b1_ref[...], 0.0)
    o_ref[...] = _dotT(h, w2_ref[...]) + b2_ref[...]


def _tc_mlp(pa, pp, w1, b1, w2, b2):
    return pl.pallas_call(
        _mlp_body,
        out_shape=jax.ShapeDtypeStruct((G, OUT), jnp.float32),
    )(pa, pp, w1, b1, w2, b2)


def kernel(x_author, x_paper, edge_index_writes, edge_index_cites,
           edge_index_self, batch_author, batch_paper,
           l1_writes_Wl, l1_writes_bl, l1_writes_Wr,
           l1_cites_Wl, l1_cites_bl, l1_cites_Wr,
           l1_self_Wl, l1_self_bl, l1_self_Wr,
           l2_writes_Wl, l2_writes_bl, l2_writes_Wr,
           l2_cites_Wl, l2_cites_bl, l2_cites_Wr,
           l2_self_Wl, l2_self_bl, l2_self_Wr,
           fc1_W, fc1_b, fc2_W, fc2_b):
    xa = jnp.pad(x_author, ((0, NA_P - N_A), (0, 64)))
    xp = jnp.pad(x_paper, ((0, NP_P - N_P), (0, 64)))

    def epad(e):
        return -(-e // EDIV) * EDIV

    src_w, dst_w = _pad_edges(edge_index_writes, epad(E_W))
    src_c, dst_c = _pad_edges(edge_index_cites, epad(E_C))
    src_s, dst_s = _pad_edges(edge_index_self, epad(E_S))

    ba3 = jnp.pad(batch_author, (0, NA_P - N_A), constant_values=G) \
        .reshape(NA_P // 1024, 1, 1024)
    bp3 = jnp.pad(batch_paper, (0, NP_P - N_P), constant_values=G) \
        .reshape(NP_P // 1024, 1, 1024)

    b2 = lambda v: v.reshape(1, -1)

    a_l, p_l = xa, xp
    for (Wlw, blw, Wrw, Wlc, blc, Wrc, Wls, bls, Wrs) in (
            (l1_writes_Wl, l1_writes_bl, l1_writes_Wr,
             l1_cites_Wl, l1_cites_bl, l1_cites_Wr,
             l1_self_Wl, l1_self_bl, l1_self_Wr),
            (l2_writes_Wl, l2_writes_bl, l2_writes_Wr,
             l2_cites_Wl, l2_cites_bl, l2_cites_Wr,
             l2_self_Wl, l2_self_bl, l2_self_Wr)):
        yw = _tc_linear(a_l, Wlw)
        yc = _tc_linear(p_l, Wlc)
        ys = _tc_linear(a_l, Wls)
        s_w = _sc_segsum(yw, src_w, dst_w, NCH_P)
        s_c = _sc_segsum(yc, src_c, dst_c, NCH_P)
        s_s = _sc_segsum(ys, src_s, dst_s, NCH_A)
        p_new = _tc_combine2(s_w, s_c, p_l, Wrw, Wrc, b2(blw), b2(blc))
        a_new = _tc_combine1(s_s, a_l, Wrs, b2(bls))
        a_l, p_l = a_new, p_new

    pa = _tc_pool(a_l, ba3)
    pp = _tc_pool(p_l, bp3)
    return _tc_mlp(pa, pp, fc1_W, b2(fc1_b), fc2_W, b2(fc2_b))


# speedup vs baseline: 5.1445x; 1.0065x over previous
import functools

import jax
import jax.numpy as jnp
from jax import lax
from jax.experimental import pallas as pl
from jax.experimental.pallas import tpu as pltpu
from jax.experimental.pallas import tpu_sc as plsc

N_A, N_P, D, H, OUT, G = 50000, 100000, 64, 64, 2, 128
E_W, E_C, E_S = 1000000, 1000000, 50000

C = 10240
RPT = C // 16
NCH_A, NCH_P = 6, 10
NA_P = NCH_A * C
NP_P = NCH_P * C
EB = 1024
EDIV = 16 * EB
BUF = 4096
RBT = BUF - EB - 128

_DNUMS = lax.GatherDimensionNumbers(
    offset_dims=(), collapsed_slice_dims=(0,), start_index_map=(0,))


def _pad_edges(ei, e_pad):
    src = jnp.pad(ei[0], (0, e_pad - ei.shape[1]))
    dst = jnp.pad(ei[1], (0, e_pad - ei.shape[1]), constant_values=-1)
    return src, dst


def _sc_segsum(y, src, dst, nchunk):
    e_pad = src.shape[0]
    nblk = e_pad // EDIV
    n_pass = nchunk // 2
    mesh = plsc.VectorSubcoreMesh(core_axis_name="c", subcore_axis_name="s")

    @functools.partial(
        pl.kernel, mesh=mesh,
        out_type=jax.ShapeDtypeStruct((nchunk * C, 128), jnp.float32),
        scratch_types=[
            pltpu.VMEM((2 * EB,), jnp.int32),
            pltpu.VMEM((2 * EB,), jnp.int32),
            pltpu.VMEM((BUF + 16,), jnp.int32),
            pltpu.VMEM((BUF + 16,), jnp.int32),
            pltpu.VMEM((128,), jnp.int32),
            pltpu.VMEM((128,), jnp.int32),
            pltpu.VMEM((256, 128), jnp.float32),
            pltpu.VMEM_SHARED((C + 16, 128), jnp.float32),
            pltpu.SemaphoreType.DMA,
            pltpu.SemaphoreType.DMA,
            pltpu.SemaphoreType.DMA,
        ],
    )
    def k(y_hbm, src_hbm, dst_hbm, out_hbm, src_v, dst_v, srcr_v, dstr_v,
          ixa_v, ixb_v, rows_v, acc_sh, sem, sem_e, sem_o):
        core = lax.axis_index("c")
        sid = lax.axis_index("s")

        ebase = sid * (e_pad // 16)
        ii16 = lax.iota(jnp.int32, 16)

        def _lg(x, idx):
            return lax.gather(
                x, idx[:, None], _DNUMS, (1,),
                mode=lax.GatherScatterMode.PROMISE_IN_BOUNDS)

        def fire(nf, tf):
            @pl.when(tf % 2 == 0)
            def _():
                for j in range(8):
                    ixa_v[pl.ds(16 * j, 16)] = \
                        dstr_v[pl.ds(nf * 128 + 16 * j, 16)]

            @pl.when(tf % 2 == 1)
            def _():
                for j in range(8):
                    ixb_v[pl.ds(16 * j, 16)] = \
                        dstr_v[pl.ds(nf * 128 + 16 * j, 16)]

            pltpu.async_copy(
                y_hbm.at[srcr_v.at[pl.ds(nf * 128, 128)]],
                rows_v.at[pl.ds((tf % 2) * 128, 128)], sem)

        def drain_prev(tf):
            par = (tf - 1) % 2
            pltpu.make_async_copy(
                y_hbm.at[pl.ds(0, 128)],
                rows_v.at[pl.ds(par * 128, 128)], sem).wait()

            @pl.when(par == 0)
            def _():
                pltpu.sync_copy(rows_v.at[pl.ds(0, 128)],
                                acc_sh.at[ixa_v], add=True)

            @pl.when(par == 1)
            def _():
                pltpu.sync_copy(rows_v.at[pl.ds(128, 128)],
                                acc_sh.at[ixb_v], add=True)

        for p in range(n_pass):
            chunk = 2 * p + core
            lo = chunk * C
            base = sid * RPT

            def zrows(i, _):
                for j in range(8):
                    rows_v[i, pl.ds(16 * j, 16)] = jnp.zeros(
                        (16,), jnp.float32)
                return 0
            lax.fori_loop(0, 256, zrows, 0)
            pltpu.sync_copy(rows_v, acc_sh.at[pl.ds(base, 256)])
            pltpu.sync_copy(rows_v, acc_sh.at[pl.ds(base + 256, 256)])
            pltpu.sync_copy(rows_v.at[pl.ds(0, 128)],
                            acc_sh.at[pl.ds(base + 512, 128)])

            @pl.when(sid == 0)
            def _():
                pltpu.sync_copy(rows_v.at[pl.ds(0, 16)],
                                acc_sh.at[pl.ds(C, 16)])

            plsc.subcore_barrier()

            pltpu.async_copy(src_hbm.at[pl.ds(ebase, EB)],
                             src_v.at[pl.ds(0, EB)], sem_e)
            pltpu.async_copy(dst_hbm.at[pl.ds(ebase, EB)],
                             dst_v.at[pl.ds(0, EB)], sem_e)

            def eblk(b, carry):
                n, nf, tf = carry
                nxt = b + 1

                @pl.when((nxt < nblk) & (nxt % 2 == 0))
                def _():
                    pltpu.async_copy(src_hbm.at[pl.ds(ebase + nxt * EB, EB)],
                                     src_v.at[pl.ds(0, EB)], sem_e)
                    pltpu.async_copy(dst_hbm.at[pl.ds(ebase + nxt * EB, EB)],
                                     dst_v.at[pl.ds(0, EB)], sem_e)

                @pl.when((nxt < nblk) & (nxt % 2 == 1))
                def _():
                    pltpu.async_copy(src_hbm.at[pl.ds(ebase + nxt * EB, EB)],
                                     src_v.at[pl.ds(EB, EB)], sem_o)
                    pltpu.async_copy(dst_hbm.at[pl.ds(ebase + nxt * EB, EB)],
                                     dst_v.at[pl.ds(EB, EB)], sem_o)

                @pl.when(b % 2 == 0)
                def _():
                    pltpu.make_async_copy(src_hbm.at[pl.ds(0, EB)],
                                          src_v.at[pl.ds(0, EB)],
                                          sem_e).wait()
                    pltpu.make_async_copy(src_hbm.at[pl.ds(0, EB)],
                                          dst_v.at[pl.ds(0, EB)],
                                          sem_e).wait()

                @pl.when(b % 2 == 1)
                def _():
                    pltpu.make_async_copy(src_hbm.at[pl.ds(0, EB)],
                                          src_v.at[pl.ds(EB, EB)],
                                          sem_o).wait()
                    pltpu.make_async_copy(src_hbm.at[pl.ds(0, EB)],
                                          dst_v.at[pl.ds(EB, EB)],
                                          sem_o).wait()

                bb = (b % 2) * EB
                def vbody(j, n):
                    u = dst_v[pl.ds(bb + 16 * j, 16)] - lo
                    m = (u >= 0) & (u < C)
                    mi = jnp.where(m, 1, 0)
                    s = mi
                    for k2 in (1, 2, 4, 8):
                        g = _lg(s, jnp.maximum(ii16 - k2, 0))
                        s = s + jnp.where(ii16 >= k2, g, 0)
                    lo2 = jnp.zeros((16,), jnp.int32)
                    for st in (8, 4, 2, 1):
                        cand = lo2 + st
                        sv = _lg(s, cand - 1)
                        lo2 = jnp.where(sv < ii16 + 1, cand, lo2)
                    lo2 = jnp.minimum(lo2, 15)
                    srcr_v[pl.ds(n, 16)] = _lg(
                        src_v[pl.ds(bb + 16 * j, 16)], lo2)
                    dstr_v[pl.ds(n, 16)] = _lg(u, lo2)
                    return n + s[15]
                n = lax.fori_loop(0, EB // 16, vbody, n, unroll=4)
                for _f in range(EB // 128):
                    @pl.when(nf + _f < n // 128)
                    def _():
                        @pl.when(tf + _f > 0)
                        def _():
                            drain_prev(tf + _f)
                        fire(nf + _f, tf + _f)
                tf = tf + (n // 128 - nf)
                nf = n // 128
                rb = n >= RBT

                @pl.when(rb)
                def _():
                    for j in range(8):
                        sv = srcr_v[pl.ds(nf * 128 + 16 * j, 16)]
                        srcr_v[pl.ds(16 * j, 16)] = sv
                        dv = dstr_v[pl.ds(nf * 128 + 16 * j, 16)]
                        dstr_v[pl.ds(16 * j, 16)] = dv
                n = jnp.where(rb, n - nf * 128, n)
                nf = jnp.where(rb, 0, nf)
                return n, nf, tf
            n, nf, tf = lax.fori_loop(
                0, nblk, eblk,
                (jnp.int32(0), jnp.int32(0), jnp.int32(0)))

            tail = n % 128

            @pl.when(tail > 0)
            def _():
                for j in range(8):
                    srcr_v[pl.ds(n + 16 * j, 16)] = \
                        jnp.zeros((16,), jnp.int32)
                    dstr_v[pl.ds(n + 16 * j, 16)] = \
                        jnp.full((16,), C, jnp.int32)

                @pl.when(tf > 0)
                def _():
                    drain_prev(tf)
                fire(nf, tf)

            @pl.when(tail > 0)
            def _():
                drain_prev(tf + 1)

            @pl.when((tail == 0) & (tf > 0))
            def _():
                drain_prev(tf)

            plsc.subcore_barrier()

            def wout(i, _):
                pltpu.sync_copy(acc_sh.at[pl.ds(base + i * 128, 128)],
                                out_hbm.at[pl.ds(lo + base + i * 128, 128)])
                return 0
            lax.fori_loop(0, 5, wout, 0)
            plsc.subcore_barrier()

    return k(y, src, dst)


def _dotT(a, b):
    return lax.dot_general(a, b, (((1,), (1,)), ((), ())),
                           preferred_element_type=jnp.float32)


def _aug(y):
    n = y.shape[0]
    return jnp.concatenate(
        [y, jnp.ones((n, 1), jnp.float32), jnp.zeros((n, 63), jnp.float32)],
        axis=1)


def _lin_body(x_ref, w_ref, o_ref):
    o_ref[...] = _aug(_dotT(x_ref[:, :64], w_ref[...]))


def _tc_linear(x, w):
    n = x.shape[0]
    return pl.pallas_call(
        _lin_body,
        grid=(n // 512,),
        in_specs=[pl.BlockSpec((512, 128), lambda i: (i, 0)),
                  pl.BlockSpec((64, 64), lambda i: (0, 0))],
        out_specs=pl.BlockSpec((512, 128), lambda i: (i, 0)),
        out_shape=jax.ShapeDtypeStruct((n, 128), jnp.float32),
    )(x, w)


def _comb2_body(sw_ref, sc_ref, x_ref, wa_ref, wb_ref, ba_ref, bb_ref,
                o_ref):
    invw = 1.0 / jnp.maximum(sw_ref[:, 64:65], 1.0)
    invc = 1.0 / jnp.maximum(sc_ref[:, 64:65], 1.0)
    w = wa_ref[...] + wb_ref[...]
    b = ba_ref[...] + bb_ref[...]
    o_ref[...] = _aug(jnp.maximum(
        sw_ref[:, :64] * invw + sc_ref[:, :64] * invc
        + _dotT(x_ref[:, :64], w) + b, 0.0))


def _tc_combine2(s_w, s_c, x, wa, wb, ba, bb):
    n = x.shape[0]
    return pl.pallas_call(
        _comb2_body,
        grid=(n // 512,),
        in_specs=[pl.BlockSpec((512, 128), lambda i: (i, 0)),
                  pl.BlockSpec((512, 128), lambda i: (i, 0)),
                  pl.BlockSpec((512, 128), lambda i: (i, 0)),
                  pl.BlockSpec((64, 64), lambda i: (0, 0)),
                  pl.BlockSpec((64, 64), lambda i: (0, 0)),
                  pl.BlockSpec((1, 64), lambda i: (0, 0)),
                  pl.BlockSpec((1, 64), lambda i: (0, 0))],
        out_specs=pl.BlockSpec((512, 128), lambda i: (i, 0)),
        out_shape=jax.ShapeDtypeStruct((n, 128), jnp.float32),
    )(s_w, s_c, x, wa, wb, ba, bb)


def _comb1_body(ss_ref, x_ref, w_ref, b_ref, o_ref):
    inv = 1.0 / jnp.maximum(ss_ref[:, 64:65], 1.0)
    o_ref[...] = _aug(jnp.maximum(
        ss_ref[:, :64] * inv + _dotT(x_ref[:, :64], w_ref[...]) + b_ref[...],
        0.0))


def _tc_combine1(s_s, x, w, b):
    n = x.shape[0]
    return pl.pallas_call(
        _comb1_body,
        grid=(n // 512,),
        in_specs=[pl.BlockSpec((512, 128), lambda i: (i, 0)),
                  pl.BlockSpec((512, 128), lambda i: (i, 0)),
                  pl.BlockSpec((64, 64), lambda i: (0, 0)),
                  pl.BlockSpec((1, 64), lambda i: (0, 0))],
        out_specs=pl.BlockSpec((512, 128), lambda i: (i, 0)),
        out_shape=jax.ShapeDtypeStruct((n, 128), jnp.float32),
    )(s_s, x, w, b)


def _pool_body(b_ref, x_ref, os_ref):
    i = pl.program_id(0)

    @pl.when(i == 0)
    def _():
        os_ref[...] = jnp.zeros_like(os_ref)

    b = b_ref[0, 0, :][None, :]
    oh = (jax.lax.broadcasted_iota(jnp.int32, (G, 1024), 0) == b
          ).astype(jnp.float32)
    os_ref[...] += lax.dot_general(oh, x_ref[...], (((1,), (0,)), ((), ())),
                                   preferred_element_type=jnp.float32)


def _tc_pool(x, b3):
    n = x.shape[0]
    return pl.pallas_call(
        _pool_body,
        grid=(n // 1024,),
        in_specs=[pl.BlockSpec((1, 1, 1024), lambda i: (i, 0, 0)),
                  pl.BlockSpec((1024, 128), lambda i: (i, 0))],
        out_specs=pl.BlockSpec((G, 128), lambda i: (0, 0)),
        out_shape=jax.ShapeDtypeStruct((G, 128), jnp.float32),
    )(b3, x)


def _mlp_body(pa_ref, pp_ref, w1_ref, b1_ref, w2_ref, b2_ref, o_ref):
    xa = pa_ref[:, :64] * (1.0 / jnp.maximum(pa_ref[:, 64:65], 1.0))
    xp = pp_ref[:, :64] * (1.0 / jnp.maximum(pp_ref[:, 64:65], 1.0))
    x = jnp.concatenate([xa, xp], axis=1)
    h = jnp.maximum(_dotT(x, w1_ref[...]) + b1_ref[...], 0.0)
    o_ref[...] = _dotT(h, w2_ref[...]) + b2_ref[...]


def _tc_mlp(pa, pp, w1, b1, w2, b2):
    return pl.pallas_call(
        _mlp_body,
        out_shape=jax.ShapeDtypeStruct((G, OUT), jnp.float32),
    )(pa, pp, w1, b1, w2, b2)


def kernel(x_author, x_paper, edge_index_writes, edge_index_cites,
           edge_index_self, batch_author, batch_paper,
           l1_writes_Wl, l1_writes_bl, l1_writes_Wr,
           l1_cites_Wl, l1_cites_bl, l1_cites_Wr,
           l1_self_Wl, l1_self_bl, l1_self_Wr,
           l2_writes_Wl, l2_writes_bl, l2_writes_Wr,
           l2_cites_Wl, l2_cites_bl, l2_cites_Wr,
           l2_self_Wl, l2_self_bl, l2_self_Wr,
           fc1_W, fc1_b, fc2_W, fc2_b):
    xa = jnp.pad(x_author, ((0, NA_P - N_A), (0, 64)))
    xp = jnp.pad(x_paper, ((0, NP_P - N_P), (0, 64)))

    def epad(e):
        return -(-e // EDIV) * EDIV

    src_w, dst_w = _pad_edges(edge_index_writes, epad(E_W))
    src_c, dst_c = _pad_edges(edge_index_cites, epad(E_C))
    src_s, dst_s = _pad_edges(edge_index_self, epad(E_S))

    ba3 = jnp.pad(batch_author, (0, NA_P - N_A), constant_values=G) \
        .reshape(NA_P // 1024, 1, 1024)
    bp3 = jnp.pad(batch_paper, (0, NP_P - N_P), constant_values=G) \
        .reshape(NP_P // 1024, 1, 1024)

    b2 = lambda v: v.reshape(1, -1)

    a_l, p_l = xa, xp
    for (Wlw, blw, Wrw, Wlc, blc, Wrc, Wls, bls, Wrs) in (
            (l1_writes_Wl, l1_writes_bl, l1_writes_Wr,
             l1_cites_Wl, l1_cites_bl, l1_cites_Wr,
             l1_self_Wl, l1_self_bl, l1_self_Wr),
            (l2_writes_Wl, l2_writes_bl, l2_writes_Wr,
             l2_cites_Wl, l2_cites_bl, l2_cites_Wr,
             l2_self_Wl, l2_self_bl, l2_self_Wr)):
        yw = _tc_linear(a_l, Wlw)
        yc = _tc_linear(p_l, Wlc)
        ys = _tc_linear(a_l, Wls)
        s_w = _sc_segsum(yw, src_w, dst_w, NCH_P)
        s_c = _sc_segsum(yc, src_c, dst_c, NCH_P)
        s_s = _sc_segsum(ys, src_s, dst_s, NCH_A)
        p_new = _tc_combine2(s_w, s_c, p_l, Wrw, Wrc, b2(blw), b2(blc))
        a_new = _tc_combine1(s_s, a_l, Wrs, b2(bls))
        a_l, p_l = a_new, p_new

    pa = _tc_pool(a_l, ba3)
    pp = _tc_pool(p_l, bp3)
    return _tc_mlp(pa, pp, fc1_W, b2(fc1_b), fc2_W, b2(fc2_b))


# speedup vs baseline: 5.6563x; 1.0995x over previous
import functools

import jax
import jax.numpy as jnp
from jax import lax
from jax.experimental import pallas as pl
from jax.experimental.pallas import tpu as pltpu
from jax.experimental.pallas import tpu_sc as plsc

N_A, N_P, D, H, OUT, G = 50000, 100000, 64, 64, 2, 128
E_W, E_C, E_S = 1000000, 1000000, 50000

C = 10240
RPT = C // 16
NCH_A, NCH_P = 6, 10
NA_P = NCH_A * C
NP_P = NCH_P * C
EB = 1024
EDIV = 16 * EB
BUF = 4096
RBT = BUF - EB - 128

_DNUMS = lax.GatherDimensionNumbers(
    offset_dims=(), collapsed_slice_dims=(0,), start_index_map=(0,))


def _pad_edges(ei, e_pad):
    src = jnp.pad(ei[0], (0, e_pad - ei.shape[1]))
    dst = jnp.pad(ei[1], (0, e_pad - ei.shape[1]), constant_values=-1)
    return src, dst


def _sc_segsum_scan(y, src, dst, nchunk):
    e_pad = src.shape[0]
    nblk = e_pad // EDIV
    n_pass = nchunk // 2
    cap = e_pad // 16 + 128
    nslot = n_pass * 32
    mesh = plsc.VectorSubcoreMesh(core_axis_name="c", subcore_axis_name="s")

    @functools.partial(
        pl.kernel, mesh=mesh,
        out_type=[
            jax.ShapeDtypeStruct((nchunk * C, 128), jnp.float32),
            jax.ShapeDtypeStruct((nslot * cap + 1024,), jnp.int32),
            jax.ShapeDtypeStruct((nslot * cap + 1024,), jnp.int32),
            jax.ShapeDtypeStruct((nslot * 16,), jnp.int32),
        ],
        scratch_types=[
            pltpu.VMEM((2 * EB,), jnp.int32),
            pltpu.VMEM((2 * EB,), jnp.int32),
            pltpu.VMEM((BUF + 16,), jnp.int32),
            pltpu.VMEM((BUF + 16,), jnp.int32),
            pltpu.VMEM((128,), jnp.int32),
            pltpu.VMEM((128,), jnp.int32),
            pltpu.VMEM((256, 128), jnp.float32),
            pltpu.VMEM((16,), jnp.int32),
            pltpu.VMEM_SHARED((C + 16, 128), jnp.float32),
            pltpu.SemaphoreType.DMA,
            pltpu.SemaphoreType.DMA,
            pltpu.SemaphoreType.DMA,
            pltpu.SemaphoreType.DMA,
        ],
    )
    def k(y_hbm, src_hbm, dst_hbm, out_hbm, srcl_hbm, dstl_hbm, cnts_hbm,
          src_v, dst_v, srcr_v, dstr_v, ixa_v, ixb_v, rows_v, cnt16_v,
          acc_sh, sem, sem_e, sem_o, semw):
        core = lax.axis_index("c")
        sid = lax.axis_index("s")

        ebase = sid * (e_pad // 16)
        ii16 = lax.iota(jnp.int32, 16)

        def _lg(x, idx):
            return lax.gather(
                x, idx[:, None], _DNUMS, (1,),
                mode=lax.GatherScatterMode.PROMISE_IN_BOUNDS)

        def fire(nf, tf, sbase):
            @pl.when(tf % 2 == 0)
            def _():
                for j in range(8):
                    ixa_v[pl.ds(16 * j, 16)] = \
                        dstr_v[pl.ds(nf * 128 + 16 * j, 16)]

            @pl.when(tf % 2 == 1)
            def _():
                for j in range(8):
                    ixb_v[pl.ds(16 * j, 16)] = \
                        dstr_v[pl.ds(nf * 128 + 16 * j, 16)]

            pltpu.async_copy(
                y_hbm.at[srcr_v.at[pl.ds(nf * 128, 128)]],
                rows_v.at[pl.ds((tf % 2) * 128, 128)], sem)
            pltpu.async_copy(srcr_v.at[pl.ds(nf * 128, 128)],
                             srcl_hbm.at[pl.ds(sbase + tf * 128, 128)], semw)
            pltpu.async_copy(dstr_v.at[pl.ds(nf * 128, 128)],
                             dstl_hbm.at[pl.ds(sbase + tf * 128, 128)], semw)

        def wait_writes(k2):
            def wb(i, _):
                pltpu.make_async_copy(
                    srcr_v.at[pl.ds(0, 128)],
                    srcl_hbm.at[pl.ds(0, 128)], semw).wait()
                return 0
            lax.fori_loop(0, k2, wb, 0)

        def drain_prev(tf):
            par = (tf - 1) % 2
            pltpu.make_async_copy(
                y_hbm.at[pl.ds(0, 128)],
                rows_v.at[pl.ds(par * 128, 128)], sem).wait()

            @pl.when(par == 0)
            def _():
                pltpu.sync_copy(rows_v.at[pl.ds(0, 128)],
                                acc_sh.at[ixa_v], add=True)

            @pl.when(par == 1)
            def _():
                pltpu.sync_copy(rows_v.at[pl.ds(128, 128)],
                                acc_sh.at[ixb_v], add=True)

        for p in range(n_pass):
            chunk = 2 * p + core
            lo = chunk * C
            base = sid * RPT
            sbase = (p * 32 + sid * 2 + core) * cap

            def zrows(i, _):
                for j in range(8):
                    rows_v[i, pl.ds(16 * j, 16)] = jnp.zeros(
                        (16,), jnp.float32)
                return 0
            lax.fori_loop(0, 256, zrows, 0)
            pltpu.sync_copy(rows_v, acc_sh.at[pl.ds(base, 256)])
            pltpu.sync_copy(rows_v, acc_sh.at[pl.ds(base + 256, 256)])
            pltpu.sync_copy(rows_v.at[pl.ds(0, 128)],
                            acc_sh.at[pl.ds(base + 512, 128)])

            @pl.when(sid == 0)
            def _():
                pltpu.sync_copy(rows_v.at[pl.ds(0, 16)],
                                acc_sh.at[pl.ds(C, 16)])

            plsc.subcore_barrier()

            pltpu.async_copy(src_hbm.at[pl.ds(ebase, EB)],
                             src_v.at[pl.ds(0, EB)], sem_e)
            pltpu.async_copy(dst_hbm.at[pl.ds(ebase, EB)],
                             dst_v.at[pl.ds(0, EB)], sem_e)

            def eblk(b, carry):
                n, nf, tf, dw = carry
                nxt = b + 1

                @pl.when((nxt < nblk) & (nxt % 2 == 0))
                def _():
                    pltpu.async_copy(src_hbm.at[pl.ds(ebase + nxt * EB, EB)],
                                     src_v.at[pl.ds(0, EB)], sem_e)
                    pltpu.async_copy(dst_hbm.at[pl.ds(ebase + nxt * EB, EB)],
                                     dst_v.at[pl.ds(0, EB)], sem_e)

                @pl.when((nxt < nblk) & (nxt % 2 == 1))
                def _():
                    pltpu.async_copy(src_hbm.at[pl.ds(ebase + nxt * EB, EB)],
                                     src_v.at[pl.ds(EB, EB)], sem_o)
                    pltpu.async_copy(dst_hbm.at[pl.ds(ebase + nxt * EB, EB)],
                                     dst_v.at[pl.ds(EB, EB)], sem_o)

                @pl.when(b % 2 == 0)
                def _():
                    pltpu.make_async_copy(src_hbm.at[pl.ds(0, EB)],
                                          src_v.at[pl.ds(0, EB)],
                                          sem_e).wait()
                    pltpu.make_async_copy(src_hbm.at[pl.ds(0, EB)],
                                          dst_v.at[pl.ds(0, EB)],
                                          sem_e).wait()

                @pl.when(b % 2 == 1)
                def _():
                    pltpu.make_async_copy(src_hbm.at[pl.ds(0, EB)],
                                          src_v.at[pl.ds(EB, EB)],
                                          sem_o).wait()
                    pltpu.make_async_copy(src_hbm.at[pl.ds(0, EB)],
                                          dst_v.at[pl.ds(EB, EB)],
                                          sem_o).wait()

                bb = (b % 2) * EB
                def vbody(j, n):
                    u = dst_v[pl.ds(bb + 16 * j, 16)] - lo
                    m = (u >= 0) & (u < C)
                    mi = jnp.where(m, 1, 0)
                    s = mi
                    for k2 in (1, 2, 4, 8):
                        g = _lg(s, jnp.maximum(ii16 - k2, 0))
                        s = s + jnp.where(ii16 >= k2, g, 0)
                    lo2 = jnp.zeros((16,), jnp.int32)
                    for st in (8, 4, 2, 1):
                        cand = lo2 + st
                        sv = _lg(s, cand - 1)
                        lo2 = jnp.where(sv < ii16 + 1, cand, lo2)
                    lo2 = jnp.minimum(lo2, 15)
                    srcr_v[pl.ds(n, 16)] = _lg(
                        src_v[pl.ds(bb + 16 * j, 16)], lo2)
                    dstr_v[pl.ds(n, 16)] = _lg(u, lo2)
                    return n + s[15]
                n = lax.fori_loop(0, EB // 16, vbody, n)
                for _f in range(EB // 128):
                    @pl.when(nf + _f < n // 128)
                    def _():
                        @pl.when(tf + _f > 0)
                        def _():
                            drain_prev(tf + _f)
                        fire(nf + _f, tf + _f, sbase)
                tf = tf + (n // 128 - nf)
                nf = n // 128
                rb = n >= RBT

                @pl.when(rb)
                def _():
                    wait_writes(2 * tf - dw)
                    for j in range(8):
                        sv = srcr_v[pl.ds(nf * 128 + 16 * j, 16)]
                        srcr_v[pl.ds(16 * j, 16)] = sv
                        dv = dstr_v[pl.ds(nf * 128 + 16 * j, 16)]
                        dstr_v[pl.ds(16 * j, 16)] = dv
                dw = jnp.where(rb, 2 * tf, dw)
                n = jnp.where(rb, n - nf * 128, n)
                nf = jnp.where(rb, 0, nf)
                return n, nf, tf, dw
            n, nf, tf, dw = lax.fori_loop(
                0, nblk, eblk,
                (jnp.int32(0), jnp.int32(0), jnp.int32(0), jnp.int32(0)))

            tail = n % 128

            @pl.when(tail > 0)
            def _():
                for j in range(8):
                    srcr_v[pl.ds(n + 16 * j, 16)] = \
                        jnp.zeros((16,), jnp.int32)
                    dstr_v[pl.ds(n + 16 * j, 16)] = \
                        jnp.full((16,), C, jnp.int32)

                @pl.when(tf > 0)
                def _():
                    drain_prev(tf)
                fire(nf, tf, sbase)

            @pl.when(tail > 0)
            def _():
                drain_prev(tf + 1)

            @pl.when((tail == 0) & (tf > 0))
            def _():
                drain_prev(tf)

            nch = tf + jnp.where(tail > 0, 1, 0)
            wait_writes(2 * nch - dw)
            cnt16_v[pl.ds(0, 16)] = jnp.full((16,), 0, jnp.int32) + nch
            pltpu.sync_copy(
                cnt16_v,
                cnts_hbm.at[pl.ds((p * 32 + sid * 2 + core) * 16, 16)])

            plsc.subcore_barrier()

            def wout(i, _):
                pltpu.sync_copy(acc_sh.at[pl.ds(base + i * 128, 128)],
                                out_hbm.at[pl.ds(lo + base + i * 128, 128)])
                return 0
            lax.fori_loop(0, 5, wout, 0)
            plsc.subcore_barrier()

    return k(y, src, dst)


def _sc_segsum_replay(y, srcl, dstl, cnts, nchunk):
    n_pass = nchunk // 2
    nslot = n_pass * 32
    cap = (srcl.shape[0] - 1024) // nslot
    mesh = plsc.VectorSubcoreMesh(core_axis_name="c", subcore_axis_name="s")

    @functools.partial(
        pl.kernel, mesh=mesh,
        out_type=jax.ShapeDtypeStruct((nchunk * C, 128), jnp.float32),
        scratch_types=[
            pltpu.VMEM((2048,), jnp.int32),
            pltpu.VMEM((2048,), jnp.int32),
            pltpu.VMEM((128,), jnp.int32),
            pltpu.VMEM((128,), jnp.int32),
            pltpu.VMEM((256, 128), jnp.float32),
            pltpu.VMEM((16,), jnp.int32),
            pltpu.VMEM_SHARED((C + 16, 128), jnp.float32),
            pltpu.SemaphoreType.DMA,
            pltpu.SemaphoreType.DMA,
        ],
    )
    def k(y_hbm, srcl_hbm, dstl_hbm, cnts_hbm, out_hbm, srcr_v, dstr_v,
          ixa_v, ixb_v, rows_v, cnt16_v, acc_sh, sem, sem_l):
        core = lax.axis_index("c")
        sid = lax.axis_index("s")

        def fire(nf, tf):
            @pl.when(tf % 2 == 0)
            def _():
                for j in range(8):
                    ixa_v[pl.ds(16 * j, 16)] = \
                        dstr_v[pl.ds(nf * 128 + 16 * j, 16)]

            @pl.when(tf % 2 == 1)
            def _():
                for j in range(8):
                    ixb_v[pl.ds(16 * j, 16)] = \
                        dstr_v[pl.ds(nf * 128 + 16 * j, 16)]

            pltpu.async_copy(
                y_hbm.at[srcr_v.at[pl.ds(nf * 128, 128)]],
                rows_v.at[pl.ds((tf % 2) * 128, 128)], sem)

        def drain_prev(tf):
            par = (tf - 1) % 2
            pltpu.make_async_copy(
                y_hbm.at[pl.ds(0, 128)],
                rows_v.at[pl.ds(par * 128, 128)], sem).wait()

            @pl.when(par == 0)
            def _():
                pltpu.sync_copy(rows_v.at[pl.ds(0, 128)],
                                acc_sh.at[ixa_v], add=True)

            @pl.when(par == 1)
            def _():
                pltpu.sync_copy(rows_v.at[pl.ds(128, 128)],
                                acc_sh.at[ixb_v], add=True)

        for p in range(n_pass):
            chunk = 2 * p + core
            lo = chunk * C
            base = sid * RPT
            sbase = (p * 32 + sid * 2 + core) * cap

            def zrows(i, _):
                for j in range(8):
                    rows_v[i, pl.ds(16 * j, 16)] = jnp.zeros(
                        (16,), jnp.float32)
                return 0
            lax.fori_loop(0, 256, zrows, 0)
            pltpu.sync_copy(rows_v, acc_sh.at[pl.ds(base, 256)])
            pltpu.sync_copy(rows_v, acc_sh.at[pl.ds(base + 256, 256)])
            pltpu.sync_copy(rows_v.at[pl.ds(0, 128)],
                            acc_sh.at[pl.ds(base + 512, 128)])

            @pl.when(sid == 0)
            def _():
                pltpu.sync_copy(rows_v.at[pl.ds(0, 16)],
                                acc_sh.at[pl.ds(C, 16)])

            plsc.subcore_barrier()

            pltpu.sync_copy(cnts_hbm.at[pl.ds((p * 32 + sid * 2 + core) * 16,
                                              16)], cnt16_v)
            nch = cnt16_v[pl.ds(0, 16)][0]
            nrun = (nch + 7) // 8

            @pl.when(nch > 0)
            def _():
                pltpu.async_copy(srcl_hbm.at[pl.ds(sbase, 1024)],
                                 srcr_v.at[pl.ds(0, 1024)], sem_l)
                pltpu.async_copy(dstl_hbm.at[pl.ds(sbase, 1024)],
                                 dstr_v.at[pl.ds(0, 1024)], sem_l)

            def run(r, tf):
                pltpu.make_async_copy(srcl_hbm.at[pl.ds(0, 1024)],
                                      srcr_v.at[pl.ds(0, 1024)],
                                      sem_l).wait()
                pltpu.make_async_copy(srcl_hbm.at[pl.ds(0, 1024)],
                                      dstr_v.at[pl.ds(0, 1024)],
                                      sem_l).wait()

                @pl.when(tf > 0)
                def _():
                    drain_prev(tf)

                @pl.when(r + 1 < nrun)
                def _():
                    par2 = ((r + 1) % 2) * 1024
                    pltpu.async_copy(
                        srcl_hbm.at[pl.ds(sbase + (r + 1) * 1024, 1024)],
                        srcr_v.at[pl.ds(par2, 1024)], sem_l)
                    pltpu.async_copy(
                        dstl_hbm.at[pl.ds(sbase + (r + 1) * 1024, 1024)],
                        dstr_v.at[pl.ds(par2, 1024)], sem_l)

                base8 = (r % 2) * 8
                for q in range(8):
                    @pl.when(r * 8 + q < nch)
                    def _():
                        @pl.when(q > 0)
                        def _():
                            drain_prev(tf + q)
                        fire(base8 + q, tf + q)
                return tf + jnp.minimum(nch - r * 8, 8)
            tf = lax.fori_loop(0, nrun, run, jnp.int32(0))

            @pl.when(tf > 0)
            def _():
                drain_prev(tf)

            plsc.subcore_barrier()

            def wout(i, _):
                pltpu.sync_copy(acc_sh.at[pl.ds(base + i * 128, 128)],
                                out_hbm.at[pl.ds(lo + base + i * 128, 128)])
                return 0
            lax.fori_loop(0, 5, wout, 0)
            plsc.subcore_barrier()

    return k(y, srcl, dstl, cnts)


def _dotT(a, b):
    return lax.dot_general(a, b, (((1,), (1,)), ((), ())),
                           preferred_element_type=jnp.float32)


def _aug(y):
    n = y.shape[0]
    return jnp.concatenate(
        [y, jnp.ones((n, 1), jnp.float32), jnp.zeros((n, 63), jnp.float32)],
        axis=1)


def _lin_body(x_ref, w_ref, o_ref):
    o_ref[...] = _aug(_dotT(x_ref[:, :64], w_ref[...]))


def _tc_linear(x, w):
    n = x.shape[0]
    return pl.pallas_call(
        _lin_body,
        grid=(n // 512,),
        in_specs=[pl.BlockSpec((512, 128), lambda i: (i, 0)),
                  pl.BlockSpec((64, 64), lambda i: (0, 0))],
        out_specs=pl.BlockSpec((512, 128), lambda i: (i, 0)),
        out_shape=jax.ShapeDtypeStruct((n, 128), jnp.float32),
    )(x, w)


def _comb2_body(sw_ref, sc_ref, x_ref, wa_ref, wb_ref, ba_ref, bb_ref,
                o_ref):
    invw = 1.0 / jnp.maximum(sw_ref[:, 64:65], 1.0)
    invc = 1.0 / jnp.maximum(sc_ref[:, 64:65], 1.0)
    w = wa_ref[...] + wb_ref[...]
    b = ba_ref[...] + bb_ref[...]
    o_ref[...] = _aug(jnp.maximum(
        sw_ref[:, :64] * invw + sc_ref[:, :64] * invc
        + _dotT(x_ref[:, :64], w) + b, 0.0))


def _tc_combine2(s_w, s_c, x, wa, wb, ba, bb):
    n = x.shape[0]
    return pl.pallas_call(
        _comb2_body,
        grid=(n // 512,),
        in_specs=[pl.BlockSpec((512, 128), lambda i: (i, 0)),
                  pl.BlockSpec((512, 128), lambda i: (i, 0)),
                  pl.BlockSpec((512, 128), lambda i: (i, 0)),
                  pl.BlockSpec((64, 64), lambda i: (0, 0)),
                  pl.BlockSpec((64, 64), lambda i: (0, 0)),
                  pl.BlockSpec((1, 64), lambda i: (0, 0)),
                  pl.BlockSpec((1, 64), lambda i: (0, 0))],
        out_specs=pl.BlockSpec((512, 128), lambda i: (i, 0)),
        out_shape=jax.ShapeDtypeStruct((n, 128), jnp.float32),
    )(s_w, s_c, x, wa, wb, ba, bb)


def _comb1_body(ss_ref, x_ref, w_ref, b_ref, o_ref):
    inv = 1.0 / jnp.maximum(ss_ref[:, 64:65], 1.0)
    o_ref[...] = _aug(jnp.maximum(
        ss_ref[:, :64] * inv + _dotT(x_ref[:, :64], w_ref[...]) + b_ref[...],
        0.0))


def _tc_combine1(s_s, x, w, b):
    n = x.shape[0]
    return pl.pallas_call(
        _comb1_body,
        grid=(n // 512,),
        in_specs=[pl.BlockSpec((512, 128), lambda i: (i, 0)),
                  pl.BlockSpec((512, 128), lambda i: (i, 0)),
                  pl.BlockSpec((64, 64), lambda i: (0, 0)),
                  pl.BlockSpec((1, 64), lambda i: (0, 0))],
        out_specs=pl.BlockSpec((512, 128), lambda i: (i, 0)),
        out_shape=jax.ShapeDtypeStruct((n, 128), jnp.float32),
    )(s_s, x, w, b)


def _pool_body(b_ref, x_ref, os_ref):
    i = pl.program_id(0)

    @pl.when(i == 0)
    def _():
        os_ref[...] = jnp.zeros_like(os_ref)

    b = b_ref[0, 0, :][None, :]
    oh = (jax.lax.broadcasted_iota(jnp.int32, (G, 1024), 0) == b
          ).astype(jnp.float32)
    os_ref[...] += lax.dot_general(oh, x_ref[...], (((1,), (0,)), ((), ())),
                                   preferred_element_type=jnp.float32)


def _tc_pool(x, b3):
    n = x.shape[0]
    return pl.pallas_call(
        _pool_body,
        grid=(n // 1024,),
        in_specs=[pl.BlockSpec((1, 1, 1024), lambda i: (i, 0, 0)),
                  pl.BlockSpec((1024, 128), lambda i: (i, 0))],
        out_specs=pl.BlockSpec((G, 128), lambda i: (0, 0)),
        out_shape=jax.ShapeDtypeStruct((G, 128), jnp.float32),
    )(b3, x)


def _mlp_body(pa_ref, pp_ref, w1_ref, b1_ref, w2_ref, b2_ref, o_ref):
    xa = pa_ref[:, :64] * (1.0 / jnp.maximum(pa_ref[:, 64:65], 1.0))
    xp = pp_ref[:, :64] * (1.0 / jnp.maximum(pp_ref[:, 64:65], 1.0))
    x = jnp.concatenate([xa, xp], axis=1)
    h = jnp.maximum(_dotT(x, w1_ref[...]) + b1_ref[...], 0.0)
    o_ref[...] = _dotT(h, w2_ref[...]) + b2_ref[...]


def _tc_mlp(pa, pp, w1, b1, w2, b2):
    return pl.pallas_call(
        _mlp_body,
        out_shape=jax.ShapeDtypeStruct((G, OUT), jnp.float32),
    )(pa, pp, w1, b1, w2, b2)


def kernel(x_author, x_paper, edge_index_writes, edge_index_cites,
           edge_index_self, batch_author, batch_paper,
           l1_writes_Wl, l1_writes_bl, l1_writes_Wr,
           l1_cites_Wl, l1_cites_bl, l1_cites_Wr,
           l1_self_Wl, l1_self_bl, l1_self_Wr,
           l2_writes_Wl, l2_writes_bl, l2_writes_Wr,
           l2_cites_Wl, l2_cites_bl, l2_cites_Wr,
           l2_self_Wl, l2_self_bl, l2_self_Wr,
           fc1_W, fc1_b, fc2_W, fc2_b):
    xa = jnp.pad(x_author, ((0, NA_P - N_A), (0, 64)))
    xp = jnp.pad(x_paper, ((0, NP_P - N_P), (0, 64)))

    def epad(e):
        return -(-e // EDIV) * EDIV

    src_w, dst_w = _pad_edges(edge_index_writes, epad(E_W))
    src_c, dst_c = _pad_edges(edge_index_cites, epad(E_C))
    src_s, dst_s = _pad_edges(edge_index_self, epad(E_S))

    ba3 = jnp.pad(batch_author, (0, NA_P - N_A), constant_values=G) \
        .reshape(NA_P // 1024, 1, 1024)
    bp3 = jnp.pad(batch_paper, (0, NP_P - N_P), constant_values=G) \
        .reshape(NP_P // 1024, 1, 1024)

    b2 = lambda v: v.reshape(1, -1)

    a_l, p_l = xa, xp
    lists = {}
    for li, (Wlw, blw, Wrw, Wlc, blc, Wrc, Wls, bls, Wrs) in enumerate((
            (l1_writes_Wl, l1_writes_bl, l1_writes_Wr,
             l1_cites_Wl, l1_cites_bl, l1_cites_Wr,
             l1_self_Wl, l1_self_bl, l1_self_Wr),
            (l2_writes_Wl, l2_writes_bl, l2_writes_Wr,
             l2_cites_Wl, l2_cites_bl, l2_cites_Wr,
             l2_self_Wl, l2_self_bl, l2_self_Wr))):
        yw = _tc_linear(a_l, Wlw)
        yc = _tc_linear(p_l, Wlc)
        ys = _tc_linear(a_l, Wls)
        if li == 0:
            s_w, *lists["w"] = _sc_segsum_scan(yw, src_w, dst_w, NCH_P)
            s_c, *lists["c"] = _sc_segsum_scan(yc, src_c, dst_c, NCH_P)
            s_s, *lists["s"] = _sc_segsum_scan(ys, src_s, dst_s, NCH_A)
        else:
            s_w = _sc_segsum_replay(yw, *lists["w"], NCH_P)
            s_c = _sc_segsum_replay(yc, *lists["c"], NCH_P)
            s_s = _sc_segsum_replay(ys, *lists["s"], NCH_A)
        p_new = _tc_combine2(s_w, s_c, p_l, Wrw, Wrc, b2(blw), b2(blc))
        a_new = _tc_combine1(s_s, a_l, Wrs, b2(bls))
        a_l, p_l = a_new, p_new

    pa = _tc_pool(a_l, ba3)
    pp = _tc_pool(p_l, bp3)
    return _tc_mlp(pa, pp, fc1_W, b2(fc1_b), fc2_W, b2(fc2_b))


# speedup vs baseline: 6.0773x; 1.0744x over previous
import functools

import jax
import jax.numpy as jnp
from jax import lax
from jax.experimental import pallas as pl
from jax.experimental.pallas import tpu as pltpu
from jax.experimental.pallas import tpu_sc as plsc

N_A, N_P, D, H, OUT, G = 50000, 100000, 64, 64, 2, 128
E_W, E_C, E_S = 1000000, 1000000, 50000

C = 10240
RPT = C // 16
NCH_A, NCH_P = 6, 10
NA_P = NCH_A * C
NP_P = NCH_P * C
EB = 1024
EDIV = 16 * EB
BUF = 4096
RBT = BUF - EB - 128

_DNUMS = lax.GatherDimensionNumbers(
    offset_dims=(), collapsed_slice_dims=(0,), start_index_map=(0,))


def _pad_edges(ei, e_pad):
    src = jnp.pad(ei[0], (0, e_pad - ei.shape[1]))
    dst = jnp.pad(ei[1], (0, e_pad - ei.shape[1]), constant_values=-1)
    return src, dst


def _sc_segsum_scan(y, src, dst, nchunk):
    e_pad = src.shape[0]
    nblk = e_pad // EDIV
    n_pass = nchunk // 2
    cap = e_pad // 16 + 128
    nslot = n_pass * 32
    mesh = plsc.VectorSubcoreMesh(core_axis_name="c", subcore_axis_name="s")

    @functools.partial(
        pl.kernel, mesh=mesh,
        out_type=[
            jax.ShapeDtypeStruct((nchunk * C, 128), jnp.float32),
            jax.ShapeDtypeStruct((nslot * cap + 1024,), jnp.int32),
            jax.ShapeDtypeStruct((nslot * cap + 1024,), jnp.int32),
            jax.ShapeDtypeStruct((nslot * 16,), jnp.int32),
        ],
        scratch_types=[
            pltpu.VMEM((2 * EB,), jnp.int32),
            pltpu.VMEM((2 * EB,), jnp.int32),
            pltpu.VMEM((BUF + 16,), jnp.int32),
            pltpu.VMEM((BUF + 16,), jnp.int32),
            pltpu.VMEM((128,), jnp.int32),
            pltpu.VMEM((128,), jnp.int32),
            pltpu.VMEM((256, 128), jnp.float32),
            pltpu.VMEM((16,), jnp.int32),
            pltpu.VMEM_SHARED((C + 16, 128), jnp.float32),
            pltpu.SemaphoreType.DMA,
            pltpu.SemaphoreType.DMA,
            pltpu.SemaphoreType.DMA,
            pltpu.SemaphoreType.DMA,
        ],
    )
    def k(y_hbm, src_hbm, dst_hbm, out_hbm, srcl_hbm, dstl_hbm, cnts_hbm,
          src_v, dst_v, srcr_v, dstr_v, ixa_v, ixb_v, rows_v, cnt16_v,
          acc_sh, sem, sem_e, sem_o, semw):
        core = lax.axis_index("c")
        sid = lax.axis_index("s")

        ebase = sid * (e_pad // 16)
        ii16 = lax.iota(jnp.int32, 16)

        def _lg(x, idx):
            return lax.gather(
                x, idx[:, None], _DNUMS, (1,),
                mode=lax.GatherScatterMode.PROMISE_IN_BOUNDS)

        def fire(nf, tf, sbase):
            @pl.when(tf % 2 == 0)
            def _():
                for j in range(8):
                    ixa_v[pl.ds(16 * j, 16)] = \
                        dstr_v[pl.ds(nf * 128 + 16 * j, 16)]

            @pl.when(tf % 2 == 1)
            def _():
                for j in range(8):
                    ixb_v[pl.ds(16 * j, 16)] = \
                        dstr_v[pl.ds(nf * 128 + 16 * j, 16)]

            pltpu.async_copy(
                y_hbm.at[srcr_v.at[pl.ds(nf * 128, 128)]],
                rows_v.at[pl.ds((tf % 2) * 128, 128)], sem)
            pltpu.async_copy(srcr_v.at[pl.ds(nf * 128, 128)],
                             srcl_hbm.at[pl.ds(sbase + tf * 128, 128)], semw)
            pltpu.async_copy(dstr_v.at[pl.ds(nf * 128, 128)],
                             dstl_hbm.at[pl.ds(sbase + tf * 128, 128)], semw)

        def wait_writes(k2):
            def wb(i, _):
                pltpu.make_async_copy(
                    srcr_v.at[pl.ds(0, 128)],
                    srcl_hbm.at[pl.ds(0, 128)], semw).wait()
                return 0
            lax.fori_loop(0, k2, wb, 0)

        def drain_prev(tf):
            par = (tf - 1) % 2
            pltpu.make_async_copy(
                y_hbm.at[pl.ds(0, 128)],
                rows_v.at[pl.ds(par * 128, 128)], sem).wait()

            @pl.when(par == 0)
            def _():
                pltpu.sync_copy(rows_v.at[pl.ds(0, 128)],
                                acc_sh.at[ixa_v], add=True)

            @pl.when(par == 1)
            def _():
                pltpu.sync_copy(rows_v.at[pl.ds(128, 128)],
                                acc_sh.at[ixb_v], add=True)

        for p in range(n_pass):
            chunk = 2 * p + core
            lo = chunk * C
            base = sid * RPT
            sbase = (p * 32 + sid * 2 + core) * cap

            def zrows(i, _):
                for j in range(8):
                    rows_v[i, pl.ds(16 * j, 16)] = jnp.zeros(
                        (16,), jnp.float32)
                return 0
            lax.fori_loop(0, 256, zrows, 0)
            pltpu.sync_copy(rows_v, acc_sh.at[pl.ds(base, 256)])
            pltpu.sync_copy(rows_v, acc_sh.at[pl.ds(base + 256, 256)])
            pltpu.sync_copy(rows_v.at[pl.ds(0, 128)],
                            acc_sh.at[pl.ds(base + 512, 128)])

            @pl.when(sid == 0)
            def _():
                pltpu.sync_copy(rows_v.at[pl.ds(0, 16)],
                                acc_sh.at[pl.ds(C, 16)])

            plsc.subcore_barrier()

            pltpu.async_copy(src_hbm.at[pl.ds(ebase, EB)],
                             src_v.at[pl.ds(0, EB)], sem_e)
            pltpu.async_copy(dst_hbm.at[pl.ds(ebase, EB)],
                             dst_v.at[pl.ds(0, EB)], sem_e)

            def eblk(b, carry):
                n, nf, tf, dw = carry
                nxt = b + 1

                @pl.when((nxt < nblk) & (nxt % 2 == 0))
                def _():
                    pltpu.async_copy(src_hbm.at[pl.ds(ebase + nxt * EB, EB)],
                                     src_v.at[pl.ds(0, EB)], sem_e)
                    pltpu.async_copy(dst_hbm.at[pl.ds(ebase + nxt * EB, EB)],
                                     dst_v.at[pl.ds(0, EB)], sem_e)

                @pl.when((nxt < nblk) & (nxt % 2 == 1))
                def _():
                    pltpu.async_copy(src_hbm.at[pl.ds(ebase + nxt * EB, EB)],
                                     src_v.at[pl.ds(EB, EB)], sem_o)
                    pltpu.async_copy(dst_hbm.at[pl.ds(ebase + nxt * EB, EB)],
                                     dst_v.at[pl.ds(EB, EB)], sem_o)

                @pl.when(b % 2 == 0)
                def _():
                    pltpu.make_async_copy(src_hbm.at[pl.ds(0, EB)],
                                          src_v.at[pl.ds(0, EB)],
                                          sem_e).wait()
                    pltpu.make_async_copy(src_hbm.at[pl.ds(0, EB)],
                                          dst_v.at[pl.ds(0, EB)],
                                          sem_e).wait()

                @pl.when(b % 2 == 1)
                def _():
                    pltpu.make_async_copy(src_hbm.at[pl.ds(0, EB)],
                                          src_v.at[pl.ds(EB, EB)],
                                          sem_o).wait()
                    pltpu.make_async_copy(src_hbm.at[pl.ds(0, EB)],
                                          dst_v.at[pl.ds(EB, EB)],
                                          sem_o).wait()

                bb = (b % 2) * EB
                def vbody(j, n):
                    u = dst_v[pl.ds(bb + 16 * j, 16)] - lo
                    m = (u >= 0) & (u < C)
                    mi = jnp.where(m, 1, 0)
                    s = mi
                    for k2 in (1, 2, 4, 8):
                        g = _lg(s, jnp.maximum(ii16 - k2, 0))
                        s = s + jnp.where(ii16 >= k2, g, 0)
                    lo2 = jnp.zeros((16,), jnp.int32)
                    for st in (8, 4, 2, 1):
                        cand = lo2 + st
                        sv = _lg(s, cand - 1)
                        lo2 = jnp.where(sv < ii16 + 1, cand, lo2)
                    lo2 = jnp.minimum(lo2, 15)
                    srcr_v[pl.ds(n, 16)] = _lg(
                        src_v[pl.ds(bb + 16 * j, 16)], lo2)
                    dstr_v[pl.ds(n, 16)] = _lg(u, lo2)
                    return n + s[15]
                n = lax.fori_loop(0, EB // 16, vbody, n)
                for _f in range(EB // 128):
                    @pl.when(nf + _f < n // 128)
                    def _():
                        fire(nf + _f, tf + _f, sbase)

                        @pl.when(tf + _f > 0)
                        def _():
                            drain_prev(tf + _f)
                tf = tf + (n // 128 - nf)
                nf = n // 128
                rb = n >= RBT

                @pl.when(rb)
                def _():
                    wait_writes(2 * tf - dw)
                    for j in range(8):
                        sv = srcr_v[pl.ds(nf * 128 + 16 * j, 16)]
                        srcr_v[pl.ds(16 * j, 16)] = sv
                        dv = dstr_v[pl.ds(nf * 128 + 16 * j, 16)]
                        dstr_v[pl.ds(16 * j, 16)] = dv
                dw = jnp.where(rb, 2 * tf, dw)
                n = jnp.where(rb, n - nf * 128, n)
                nf = jnp.where(rb, 0, nf)
                return n, nf, tf, dw
            n, nf, tf, dw = lax.fori_loop(
                0, nblk, eblk,
                (jnp.int32(0), jnp.int32(0), jnp.int32(0), jnp.int32(0)))

            tail = n % 128

            @pl.when(tail > 0)
            def _():
                for j in range(8):
                    srcr_v[pl.ds(n + 16 * j, 16)] = \
                        jnp.zeros((16,), jnp.int32)
                    dstr_v[pl.ds(n + 16 * j, 16)] = \
                        jnp.full((16,), C, jnp.int32)

                fire(nf, tf, sbase)

                @pl.when(tf > 0)
                def _():
                    drain_prev(tf)

            @pl.when(tail > 0)
            def _():
                drain_prev(tf + 1)

            @pl.when((tail == 0) & (tf > 0))
            def _():
                drain_prev(tf)

            nch = tf + jnp.where(tail > 0, 1, 0)
            wait_writes(2 * nch - dw)
            cnt16_v[pl.ds(0, 16)] = jnp.full((16,), 0, jnp.int32) + nch
            pltpu.sync_copy(
                cnt16_v,
                cnts_hbm.at[pl.ds((p * 32 + sid * 2 + core) * 16, 16)])

            plsc.subcore_barrier()

            def wout(i, _):
                pltpu.sync_copy(acc_sh.at[pl.ds(base + i * 128, 128)],
                                out_hbm.at[pl.ds(lo + base + i * 128, 128)])
                return 0
            lax.fori_loop(0, 5, wout, 0)
            plsc.subcore_barrier()

    return k(y, src, dst)


def _sc_segsum_replay(y, srcl, dstl, cnts, nchunk):
    n_pass = nchunk // 2
    nslot = n_pass * 32
    cap = (srcl.shape[0] - 1024) // nslot
    mesh = plsc.VectorSubcoreMesh(core_axis_name="c", subcore_axis_name="s")

    @functools.partial(
        pl.kernel, mesh=mesh,
        out_type=jax.ShapeDtypeStruct((nchunk * C, 128), jnp.float32),
        scratch_types=[
            pltpu.VMEM((2048,), jnp.int32),
            pltpu.VMEM((2048,), jnp.int32),
            pltpu.VMEM((128,), jnp.int32),
            pltpu.VMEM((128,), jnp.int32),
            pltpu.VMEM((256, 128), jnp.float32),
            pltpu.VMEM((16,), jnp.int32),
            pltpu.VMEM_SHARED((C + 16, 128), jnp.float32),
            pltpu.SemaphoreType.DMA,
            pltpu.SemaphoreType.DMA,
        ],
    )
    def k(y_hbm, srcl_hbm, dstl_hbm, cnts_hbm, out_hbm, srcr_v, dstr_v,
          ixa_v, ixb_v, rows_v, cnt16_v, acc_sh, sem, sem_l):
        core = lax.axis_index("c")
        sid = lax.axis_index("s")

        def fire(nf, tf):
            @pl.when(tf % 2 == 0)
            def _():
                for j in range(8):
                    ixa_v[pl.ds(16 * j, 16)] = \
                        dstr_v[pl.ds(nf * 128 + 16 * j, 16)]

            @pl.when(tf % 2 == 1)
            def _():
                for j in range(8):
                    ixb_v[pl.ds(16 * j, 16)] = \
                        dstr_v[pl.ds(nf * 128 + 16 * j, 16)]

            pltpu.async_copy(
                y_hbm.at[srcr_v.at[pl.ds(nf * 128, 128)]],
                rows_v.at[pl.ds((tf % 2) * 128, 128)], sem)

        def drain_prev(tf):
            par = (tf - 1) % 2
            pltpu.make_async_copy(
                y_hbm.at[pl.ds(0, 128)],
                rows_v.at[pl.ds(par * 128, 128)], sem).wait()

            @pl.when(par == 0)
            def _():
                pltpu.sync_copy(rows_v.at[pl.ds(0, 128)],
                                acc_sh.at[ixa_v], add=True)

            @pl.when(par == 1)
            def _():
                pltpu.sync_copy(rows_v.at[pl.ds(128, 128)],
                                acc_sh.at[ixb_v], add=True)

        for p in range(n_pass):
            chunk = 2 * p + core
            lo = chunk * C
            base = sid * RPT
            sbase = (p * 32 + sid * 2 + core) * cap

            def zrows(i, _):
                for j in range(8):
                    rows_v[i, pl.ds(16 * j, 16)] = jnp.zeros(
                        (16,), jnp.float32)
                return 0
            lax.fori_loop(0, 256, zrows, 0)
            pltpu.sync_copy(rows_v, acc_sh.at[pl.ds(base, 256)])
            pltpu.sync_copy(rows_v, acc_sh.at[pl.ds(base + 256, 256)])
            pltpu.sync_copy(rows_v.at[pl.ds(0, 128)],
                            acc_sh.at[pl.ds(base + 512, 128)])

            @pl.when(sid == 0)
            def _():
                pltpu.sync_copy(rows_v.at[pl.ds(0, 16)],
                                acc_sh.at[pl.ds(C, 16)])

            plsc.subcore_barrier()

            pltpu.sync_copy(cnts_hbm.at[pl.ds((p * 32 + sid * 2 + core) * 16,
                                              16)], cnt16_v)
            nch = cnt16_v[pl.ds(0, 16)][0]
            nrun = (nch + 7) // 8

            @pl.when(nch > 0)
            def _():
                pltpu.async_copy(srcl_hbm.at[pl.ds(sbase, 1024)],
                                 srcr_v.at[pl.ds(0, 1024)], sem_l)
                pltpu.async_copy(dstl_hbm.at[pl.ds(sbase, 1024)],
                                 dstr_v.at[pl.ds(0, 1024)], sem_l)

            def run(r, tf):
                pltpu.make_async_copy(srcl_hbm.at[pl.ds(0, 1024)],
                                      srcr_v.at[pl.ds(0, 1024)],
                                      sem_l).wait()
                pltpu.make_async_copy(srcl_hbm.at[pl.ds(0, 1024)],
                                      dstr_v.at[pl.ds(0, 1024)],
                                      sem_l).wait()

                @pl.when(tf > 0)
                def _():
                    drain_prev(tf)

                @pl.when(r + 1 < nrun)
                def _():
                    par2 = ((r + 1) % 2) * 1024
                    pltpu.async_copy(
                        srcl_hbm.at[pl.ds(sbase + (r + 1) * 1024, 1024)],
                        srcr_v.at[pl.ds(par2, 1024)], sem_l)
                    pltpu.async_copy(
                        dstl_hbm.at[pl.ds(sbase + (r + 1) * 1024, 1024)],
                        dstr_v.at[pl.ds(par2, 1024)], sem_l)

                base8 = (r % 2) * 8
                for q in range(8):
                    @pl.when(r * 8 + q < nch)
                    def _():
                        fire(base8 + q, tf + q)

                        @pl.when(q > 0)
                        def _():
                            drain_prev(tf + q)
                return tf + jnp.minimum(nch - r * 8, 8)
            tf = lax.fori_loop(0, nrun, run, jnp.int32(0))

            @pl.when(tf > 0)
            def _():
                drain_prev(tf)

            plsc.subcore_barrier()

            def wout(i, _):
                pltpu.sync_copy(acc_sh.at[pl.ds(base + i * 128, 128)],
                                out_hbm.at[pl.ds(lo + base + i * 128, 128)])
                return 0
            lax.fori_loop(0, 5, wout, 0)
            plsc.subcore_barrier()

    return k(y, srcl, dstl, cnts)


def _dotT(a, b):
    return lax.dot_general(a, b, (((1,), (1,)), ((), ())),
                           preferred_element_type=jnp.float32)


def _aug(y):
    n = y.shape[0]
    return jnp.concatenate(
        [y, jnp.ones((n, 1), jnp.float32), jnp.zeros((n, 63), jnp.float32)],
        axis=1)


def _lin_body(x_ref, w_ref, o_ref):
    o_ref[...] = _aug(_dotT(x_ref[:, :64], w_ref[...]))


def _tc_linear(x, w):
    n = x.shape[0]
    return pl.pallas_call(
        _lin_body,
        grid=(n // 512,),
        in_specs=[pl.BlockSpec((512, 128), lambda i: (i, 0)),
                  pl.BlockSpec((64, 64), lambda i: (0, 0))],
        out_specs=pl.BlockSpec((512, 128), lambda i: (i, 0)),
        out_shape=jax.ShapeDtypeStruct((n, 128), jnp.float32),
    )(x, w)


def _comb2_body(sw_ref, sc_ref, x_ref, wa_ref, wb_ref, ba_ref, bb_ref,
                o_ref):
    invw = 1.0 / jnp.maximum(sw_ref[:, 64:65], 1.0)
    invc = 1.0 / jnp.maximum(sc_ref[:, 64:65], 1.0)
    w = wa_ref[...] + wb_ref[...]
    b = ba_ref[...] + bb_ref[...]
    o_ref[...] = _aug(jnp.maximum(
        sw_ref[:, :64] * invw + sc_ref[:, :64] * invc
        + _dotT(x_ref[:, :64], w) + b, 0.0))


def _tc_combine2(s_w, s_c, x, wa, wb, ba, bb):
    n = x.shape[0]
    return pl.pallas_call(
        _comb2_body,
        grid=(n // 512,),
        in_specs=[pl.BlockSpec((512, 128), lambda i: (i, 0)),
                  pl.BlockSpec((512, 128), lambda i: (i, 0)),
                  pl.BlockSpec((512, 128), lambda i: (i, 0)),
                  pl.BlockSpec((64, 64), lambda i: (0, 0)),
                  pl.BlockSpec((64, 64), lambda i: (0, 0)),
                  pl.BlockSpec((1, 64), lambda i: (0, 0)),
                  pl.BlockSpec((1, 64), lambda i: (0, 0))],
        out_specs=pl.BlockSpec((512, 128), lambda i: (i, 0)),
        out_shape=jax.ShapeDtypeStruct((n, 128), jnp.float32),
    )(s_w, s_c, x, wa, wb, ba, bb)


def _comb1_body(ss_ref, x_ref, w_ref, b_ref, o_ref):
    inv = 1.0 / jnp.maximum(ss_ref[:, 64:65], 1.0)
    o_ref[...] = _aug(jnp.maximum(
        ss_ref[:, :64] * inv + _dotT(x_ref[:, :64], w_ref[...]) + b_ref[...],
        0.0))


def _tc_combine1(s_s, x, w, b):
    n = x.shape[0]
    return pl.pallas_call(
        _comb1_body,
        grid=(n // 512,),
        in_specs=[pl.BlockSpec((512, 128), lambda i: (i, 0)),
                  pl.BlockSpec((512, 128), lambda i: (i, 0)),
                  pl.BlockSpec((64, 64), lambda i: (0, 0)),
                  pl.BlockSpec((1, 64), lambda i: (0, 0))],
        out_specs=pl.BlockSpec((512, 128), lambda i: (i, 0)),
        out_shape=jax.ShapeDtypeStruct((n, 128), jnp.float32),
    )(s_s, x, w, b)


def _pool_body(b_ref, x_ref, os_ref):
    i = pl.program_id(0)

    @pl.when(i == 0)
    def _():
        os_ref[...] = jnp.zeros_like(os_ref)

    b = b_ref[0, 0, :][None, :]
    oh = (jax.lax.broadcasted_iota(jnp.int32, (G, 1024), 0) == b
          ).astype(jnp.float32)
    os_ref[...] += lax.dot_general(oh, x_ref[...], (((1,), (0,)), ((), ())),
                                   preferred_element_type=jnp.float32)


def _tc_pool(x, b3):
    n = x.shape[0]
    return pl.pallas_call(
        _pool_body,
        grid=(n // 1024,),
        in_specs=[pl.BlockSpec((1, 1, 1024), lambda i: (i, 0, 0)),
                  pl.BlockSpec((1024, 128), lambda i: (i, 0))],
        out_specs=pl.BlockSpec((G, 128), lambda i: (0, 0)),
        out_shape=jax.ShapeDtypeStruct((G, 128), jnp.float32),
    )(b3, x)


def _mlp_body(pa_ref, pp_ref, w1_ref, b1_ref, w2_ref, b2_ref, o_ref):
    xa = pa_ref[:, :64] * (1.0 / jnp.maximum(pa_ref[:, 64:65], 1.0))
    xp = pp_ref[:, :64] * (1.0 / jnp.maximum(pp_ref[:, 64:65], 1.0))
    x = jnp.concatenate([xa, xp], axis=1)
    h = jnp.maximum(_dotT(x, w1_ref[...]) + b1_ref[...], 0.0)
    o_ref[...] = _dotT(h, w2_ref[...]) + b2_ref[...]


def _tc_mlp(pa, pp, w1, b1, w2, b2):
    return pl.pallas_call(
        _mlp_body,
        out_shape=jax.ShapeDtypeStruct((G, OUT), jnp.float32),
    )(pa, pp, w1, b1, w2, b2)


def kernel(x_author, x_paper, edge_index_writes, edge_index_cites,
           edge_index_self, batch_author, batch_paper,
           l1_writes_Wl, l1_writes_bl, l1_writes_Wr,
           l1_cites_Wl, l1_cites_bl, l1_cites_Wr,
           l1_self_Wl, l1_self_bl, l1_self_Wr,
           l2_writes_Wl, l2_writes_bl, l2_writes_Wr,
           l2_cites_Wl, l2_cites_bl, l2_cites_Wr,
           l2_self_Wl, l2_self_bl, l2_self_Wr,
           fc1_W, fc1_b, fc2_W, fc2_b):
    xa = jnp.pad(x_author, ((0, NA_P - N_A), (0, 64)))
    xp = jnp.pad(x_paper, ((0, NP_P - N_P), (0, 64)))

    def epad(e):
        return -(-e // EDIV) * EDIV

    src_w, dst_w = _pad_edges(edge_index_writes, epad(E_W))
    src_c, dst_c = _pad_edges(edge_index_cites, epad(E_C))
    src_s, dst_s = _pad_edges(edge_index_self, epad(E_S))

    ba3 = jnp.pad(batch_author, (0, NA_P - N_A), constant_values=G) \
        .reshape(NA_P // 1024, 1, 1024)
    bp3 = jnp.pad(batch_paper, (0, NP_P - N_P), constant_values=G) \
        .reshape(NP_P // 1024, 1, 1024)

    b2 = lambda v: v.reshape(1, -1)

    a_l, p_l = xa, xp
    lists = {}
    for li, (Wlw, blw, Wrw, Wlc, blc, Wrc, Wls, bls, Wrs) in enumerate((
            (l1_writes_Wl, l1_writes_bl, l1_writes_Wr,
             l1_cites_Wl, l1_cites_bl, l1_cites_Wr,
             l1_self_Wl, l1_self_bl, l1_self_Wr),
            (l2_writes_Wl, l2_writes_bl, l2_writes_Wr,
             l2_cites_Wl, l2_cites_bl, l2_cites_Wr,
             l2_self_Wl, l2_self_bl, l2_self_Wr))):
        yw = _tc_linear(a_l, Wlw)
        yc = _tc_linear(p_l, Wlc)
        ys = _tc_linear(a_l, Wls)
        if li == 0:
            s_w, *lists["w"] = _sc_segsum_scan(yw, src_w, dst_w, NCH_P)
            s_c, *lists["c"] = _sc_segsum_scan(yc, src_c, dst_c, NCH_P)
            s_s, *lists["s"] = _sc_segsum_scan(ys, src_s, dst_s, NCH_A)
        else:
            s_w = _sc_segsum_replay(yw, *lists["w"], NCH_P)
            s_c = _sc_segsum_replay(yc, *lists["c"], NCH_P)
            s_s = _sc_segsum_replay(ys, *lists["s"], NCH_A)
        p_new = _tc_combine2(s_w, s_c, p_l, Wrw, Wrc, b2(blw), b2(blc))
        a_new = _tc_combine1(s_s, a_l, Wrs, b2(bls))
        a_l, p_l = a_new, p_new

    pa = _tc_pool(a_l, ba3)
    pp = _tc_pool(p_l, bp3)
    return _tc_mlp(pa, pp, fc1_W, b2(fc1_b), fc2_W, b2(fc2_b))
